# baseline (speedup 1.0000x reference)
import functools

import jax
import jax.numpy as jnp
import numpy as np
from jax import lax
from jax.experimental import pallas as pl
from jax.experimental.pallas import tpu as pltpu
from jax.experimental.pallas import tpu_sc as plsc

N = 10000
E = 320000
CS = 128
CZ = 128
H = 8
CH = 16
PQK = 4
PV = 8

DQP = H * PQK * 3
DVP = H * PV * 3
DDST = CS + DQP
DGA = CS + DQP + CS
DGB = DVP
DREC = 16
DA1 = 144
DA2 = 192

NSUB = 16
EPT = 20096
EPAD = NSUB * EPT
K1 = 32
K2 = 64
NC1 = EPT // K1
NC2 = EPT // K2
HALF = 5000
RPC = 5120
ROWS_PT = RPC // NSUB

BN_A = 1000
BN_B = 8000


def _proj_body(nf, xt, wq, wk, wv, wqp, wkp, wvp, gdst, ga, gb):
    x = nf[...]
    xq = xt[:, :DQP]
    q = jnp.dot(x, wq[...], preferred_element_type=jnp.float32)
    qp = jnp.dot(x, wqp[...], preferred_element_type=jnp.float32) + xq
    gdst[...] = jnp.concatenate([q, qp], axis=-1)
    k = jnp.dot(x, wk[...], preferred_element_type=jnp.float32)
    kp = jnp.dot(x, wkp[...], preferred_element_type=jnp.float32) + xq
    v = jnp.dot(x, wv[...], preferred_element_type=jnp.float32)
    ga[...] = jnp.concatenate([k, kp, v], axis=-1)
    gb[...] = jnp.dot(x, wvp[...], preferred_element_type=jnp.float32) + xt[...]


def _projections(nf, xt, wq, wk, wv, wqp, wkp, wvp):
    grid = (N // BN_A,)
    row = lambda i: (i, 0)
    full = lambda i: (0, 0)
    return pl.pallas_call(
        _proj_body,
        grid=grid,
        in_specs=[
            pl.BlockSpec((BN_A, CS), row),
            pl.BlockSpec((BN_A, DVP), row),
            pl.BlockSpec((CS, CS), full),
            pl.BlockSpec((CS, CS), full),
            pl.BlockSpec((CS, CS), full),
            pl.BlockSpec((CS, DQP), full),
            pl.BlockSpec((CS, DQP), full),
            pl.BlockSpec((CS, DVP), full),
        ],
        out_specs=[
            pl.BlockSpec((BN_A, DDST), row),
            pl.BlockSpec((BN_A, DGA), row),
            pl.BlockSpec((BN_A, DGB), row),
        ],
        out_shape=[
            jax.ShapeDtypeStruct((N, DDST), jnp.float32),
            jax.ShapeDtypeStruct((N, DGA), jnp.float32),
            jax.ShapeDtypeStruct((N, DGB), jnp.float32),
        ],
    )(nf, xt, wq, wk, wv, wqp, wkp, wvp)


def _rec_body(eib, ef, wb, out):
    b = jnp.dot(ef[...], wb[...], preferred_element_type=jnp.float32)
    z = jnp.zeros((BN_B, 6), jnp.float32)
    out[...] = jnp.concatenate([eib[...], z, b], axis=-1)


def _edge_record(eib, ef, wb):
    grid = (E // BN_B,)
    return pl.pallas_call(
        _rec_body,
        grid=grid,
        in_specs=[
            pl.BlockSpec((BN_B, 2), lambda i: (i, 0)),
            pl.BlockSpec((BN_B, CZ), lambda i: (i, 0)),
            pl.BlockSpec((CZ, H), lambda i: (0, 0)),
        ],
        out_specs=pl.BlockSpec((BN_B, DREC), lambda i: (i, 0)),
        out_shape=jax.ShapeDtypeStruct((E, DREC), jnp.float32),
    )(eib, ef, wb)


def _sc_mesh():
    return plsc.VectorSubcoreMesh(core_axis_name="c", subcore_axis_name="s")


def _zero_acc(ct, acc_sh, s_idx, kc):
    zero16 = jnp.zeros((16,), jnp.float32)
    width = ct.shape[1]
    for r in range(kc):
        for cc in range(width // 16):
            ct[r, pl.ds(cc * 16, 16)] = zero16
    for off in range(0, ROWS_PT, kc):
        sz = min(kc, ROWS_PT - off)
        pltpu.sync_copy(ct.at[pl.ds(0, sz)],
                        acc_sh.at[pl.ds(s_idx * ROWS_PT + off, sz)])


def _copy_out(acc_sh, out_hbm, c_idx, s_idx):
    pltpu.sync_copy(acc_sh.at[pl.ds(s_idx * ROWS_PT, ROWS_PT)],
                    out_hbm.at[c_idx, pl.ds(s_idx * ROWS_PT, ROWS_PT)])


def _c1_body(rec_hbm, gdst_hbm, ga_hbm, acc_hbm, ex_hbm,
             rec0, rec1, gd0, gd1, ga0, ga1, ct0, ct1, exv0, exv1,
             srcv0, srcv1, dstv0, dstv1, idxm0, idxm1, acc_sh,
             srec0, srec1, sgd0, sgd1, sga0, sga1, ssc0, ssc1, sex0, sex1):
    c_idx = lax.axis_index("c")
    s_idx = lax.axis_index("s")
    lo = c_idx * HALF
    iot = lax.iota(jnp.int32, 16)
    base = s_idx * EPT
    is0 = c_idx == 0

    rec = (rec0, rec1)
    gd = (gd0, gd1)
    ga = (ga0, ga1)
    ct = (ct0, ct1)
    exv = (exv0, exv1)
    srcv = (srcv0, srcv1)
    dstv = (dstv0, dstv1)
    idxm = (idxm0, idxm1)
    srec = (srec0, srec1)
    sgd = (sgd0, sgd1)
    sga = (sga0, sga1)
    ssc = (ssc0, ssc1)
    sex = (sex0, sex1)

    _zero_acc(ct0, acc_sh, s_idx, K1)
    plsc.subcore_barrier()

    def fetch_rec(h, s):
        pltpu.async_copy(rec_hbm.at[pl.ds(base + h * K1, K1)], rec[s], srec[s])

    def extract(h, s):
        e0 = base + h * K1
        for g in range(K1 // 16):
            r = g * 16 + iot
            sf = plsc.load_gather(rec[s], [r, jnp.zeros((16,), jnp.int32)])
            df = plsc.load_gather(rec[s], [r, jnp.full((16,), 1, jnp.int32)])
            sv = plsc.bitcast(sf, jnp.int32)
            dv = plsc.bitcast(df, jnp.int32)
            srcv[s][pl.ds(g * 16, 16)] = sv
            dstv[s][pl.ds(g * 16, 16)] = dv
            loc = dv - lo
            ok = (loc >= 0) & (loc < HALF) & ((iot + (e0 + g * 16)) < E)
            idxm[s][pl.ds(g * 16, 16)] = jnp.where(ok, loc, HALF)
        pltpu.async_copy(gdst_hbm.at[dstv[s]], gd[s], sgd[s])
        pltpu.async_copy(ga_hbm.at[srcv[s]], ga[s], sga[s])

    def compute_pair(h0):
        zc = tuple(lax.shift_right_arithmetic(srcv[s][pl.ds(0, 16)], 31)
                   for s in (0, 1))

        def head_body(hh, carry):
            for s in (0, 1):
                zcol = zc[s]
                cqk0 = zcol + hh * CH
                cp0 = zcol + (CS + hh * 12)
                cv0 = zcol + (CS + DQP + hh * CH)
                co0 = zcol + (16 + hh * CH)
                for g in range(K1 // 16):
                    r = g * 16 + iot
                    lqa = (plsc.load_gather(gd[s], [r, cqk0])
                           * plsc.load_gather(ga[s], [r, cqk0]))
                    lqb = (plsc.load_gather(gd[s], [r, cqk0 + 1])
                           * plsc.load_gather(ga[s], [r, cqk0 + 1]))
                    for cc in range(2, CH, 2):
                        lqa = lqa + (plsc.load_gather(gd[s], [r, cqk0 + cc])
                                     * plsc.load_gather(ga[s], [r, cqk0 + cc]))
                        lqb = lqb + (plsc.load_gather(gd[s], [r, cqk0 + cc + 1])
                                     * plsc.load_gather(ga[s], [r, cqk0 + cc + 1]))
                    d0 = (plsc.load_gather(gd[s], [r, cp0])
                          - plsc.load_gather(ga[s], [r, cp0]))
                    d1 = (plsc.load_gather(gd[s], [r, cp0 + 1])
                          - plsc.load_gather(ga[s], [r, cp0 + 1]))
                    pda = d0 * d0
                    pdb = d1 * d1
                    for cc in range(2, 12, 2):
                        da = (plsc.load_gather(gd[s], [r, cp0 + cc])
                              - plsc.load_gather(ga[s], [r, cp0 + cc]))
                        pda = pda + da * da
                        db = (plsc.load_gather(gd[s], [r, cp0 + cc + 1])
                              - plsc.load_gather(ga[s], [r, cp0 + cc + 1]))
                        pdb = pdb + db * db
                    bh = plsc.load_gather(rec[s], [r, zcol + (8 + hh)])
                    ex = jnp.exp((lqa + lqb) * 0.25 + bh - 0.1 * (pda + pdb))
                    plsc.store_scatter(ct[s], [r, zcol + hh], ex)
                    plsc.store_scatter(exv[s], [r, zcol + hh], ex)
                    for cc in range(CH):
                        plsc.store_scatter(ct[s], [r, co0 + cc],
                                           ex * plsc.load_gather(ga[s], [r, cv0 + cc]))
            return carry

        lax.fori_loop(0, H, head_body, 0)
        for s in (0, 1):
            e0 = base + (h0 + s) * K1
            pltpu.async_copy(ct[s], acc_sh.at[idxm[s]], ssc[s], add=True)

            @pl.when(is0)
            def _(s=s, e0=e0):
                pltpu.async_copy(exv[s], ex_hbm.at[pl.ds(e0, K1)], sex[s])

    fetch_rec(0, 0)
    fetch_rec(1, 1)

    npair = NC1 // 2

    def body(p, carry):
        h0 = 2 * p
        for s in (0, 1):
            h = h0 + s

            @pl.when(p > 0)
            def _(s=s):
                pltpu.make_async_copy(ct[s], acc_sh.at[idxm[s]], ssc[s]).wait()

                @pl.when(is0)
                def _():
                    pltpu.make_async_copy(exv[s], ex_hbm.at[pl.ds(0, K1)],
                                          sex[s]).wait()

            pltpu.make_async_copy(rec_hbm.at[pl.ds(0, K1)], rec[s], srec[s]).wait()
            extract(h, s)
        for s in (0, 1):
            pltpu.make_async_copy(gdst_hbm.at[dstv[s]], gd[s], sgd[s]).wait()
            pltpu.make_async_copy(ga_hbm.at[srcv[s]], ga[s], sga[s]).wait()
        compute_pair(h0)
        for s in (0, 1):

            @pl.when(p < npair - 1)
            def _(h=h0 + s, s=s):
                fetch_rec(h + 2, s)

        return carry

    lax.fori_loop(0, npair, body, 0)
    for s in (0, 1):
        pltpu.make_async_copy(ct[s], acc_sh.at[idxm[s]], ssc[s]).wait()

        @pl.when(is0)
        def _(s=s):
            pltpu.make_async_copy(exv[s], ex_hbm.at[pl.ds(0, K1)], sex[s]).wait()

    plsc.subcore_barrier()
    _copy_out(acc_sh, acc_hbm, c_idx, s_idx)


def _c2_body(rec_hbm, ex_hbm, gb_hbm, acc_hbm,
             rec0, rec1, exv0, exv1, gb0, gb1, ct0, ct1,
             srcv0, srcv1, dstv0, dstv1, idxm0, idxm1, acc_sh,
             srec0, srec1, sev0, sev1, sgb0, sgb1, ssc0, ssc1):
    c_idx = lax.axis_index("c")
    s_idx = lax.axis_index("s")
    lo = c_idx * HALF
    iot = lax.iota(jnp.int32, 16)
    base = s_idx * EPT

    rec = (rec0, rec1)
    exv = (exv0, exv1)
    gb = (gb0, gb1)
    ct = (ct0, ct1)
    srcv = (srcv0, srcv1)
    dstv = (dstv0, dstv1)
    idxm = (idxm0, idxm1)
    srec = (srec0, srec1)
    sev = (sev0, sev1)
    sgb = (sgb0, sgb1)
    ssc = (ssc0, ssc1)

    _zero_acc(ct0, acc_sh, s_idx, K2)
    plsc.subcore_barrier()

    def fetch(h, s):
        pltpu.async_copy(rec_hbm.at[pl.ds(base + h * K2, K2)], rec[s], srec[s])
        pltpu.async_copy(ex_hbm.at[pl.ds(base + h * K2, K2)], exv[s], sev[s])

    def extract(h, s):
        e0 = base + h * K2
        for g in range(K2 // 16):
            r = g * 16 + iot
            df = plsc.load_gather(rec[s], [r, jnp.full((16,), 1, jnp.int32)])
            sf = plsc.load_gather(rec[s], [r, jnp.zeros((16,), jnp.int32)])
            dv = plsc.bitcast(df, jnp.int32)
            srcv[s][pl.ds(g * 16, 16)] = plsc.bitcast(sf, jnp.int32)
            loc = dv - lo
            ok = (loc >= 0) & (loc < HALF) & ((iot + (e0 + g * 16)) < E)
            idxm[s][pl.ds(g * 16, 16)] = jnp.where(ok, loc, HALF)
        pltpu.async_copy(gb_hbm.at[srcv[s]], gb[s], sgb[s])

    def compute_pair():
        zc = tuple(lax.shift_right_arithmetic(srcv[s][pl.ds(0, 16)], 31)
                   for s in (0, 1))

        def head_body(hh, carry):
            for s in (0, 1):
                zcol = zc[s]
                cb = zcol + hh * 24
                for g in range(K2 // 16):
                    r = g * 16 + iot
                    ex = plsc.load_gather(exv[s], [r, zcol + hh])
                    for cc in range(24):
                        plsc.store_scatter(ct[s], [r, cb + cc],
                                           ex * plsc.load_gather(gb[s], [r, cb + cc]))
            return carry

        lax.fori_loop(0, H, head_body, 0)
        for s in (0, 1):
            pltpu.async_copy(ct[s], acc_sh.at[idxm[s]], ssc[s], add=True)

    fetch(0, 0)
    fetch(1, 1)
    npair = NC2 // 2

    def body(p, carry):
        h0 = 2 * p
        for s in (0, 1):
            h = h0 + s

            @pl.when(p > 0)
            def _(s=s):
                pltpu.make_async_copy(ct[s], acc_sh.at[idxm[s]], ssc[s]).wait()

            pltpu.make_async_copy(rec_hbm.at[pl.ds(0, K2)], rec[s], srec[s]).wait()
            extract(h, s)
        for s in (0, 1):
            pltpu.make_async_copy(ex_hbm.at[pl.ds(0, K2)], exv[s], sev[s]).wait()
            pltpu.make_async_copy(gb_hbm.at[srcv[s]], gb[s], sgb[s]).wait()
        compute_pair()
        for s in (0, 1):

            @pl.when(p < npair - 1)
            def _(h=h0 + s, s=s):
                fetch(h + 2, s)

        return carry

    lax.fori_loop(0, npair, body, 0)
    for s in (0, 1):
        pltpu.make_async_copy(ct[s], acc_sh.at[idxm[s]], ssc[s]).wait()
    plsc.subcore_barrier()
    _copy_out(acc_sh, acc_hbm, c_idx, s_idx)


def _edge_phase_sc(rec, gdst, ga, gb):
    params = pltpu.CompilerParams(use_tc_tiling_on_sc=False,
                                  needs_layout_passes=False)
    c1 = functools.partial(
        pl.kernel,
        out_type=[jax.ShapeDtypeStruct((2, RPC, DA1), jnp.float32),
                  jax.ShapeDtypeStruct((EPAD, H), jnp.float32)],
        mesh=_sc_mesh(),
        compiler_params=params,
        scratch_types=(
            [pltpu.VMEM((K1, DREC), jnp.float32)] * 2
            + [pltpu.VMEM((K1, DDST), jnp.float32)] * 2
            + [pltpu.VMEM((K1, DGA), jnp.float32)] * 2
            + [pltpu.VMEM((K1, DA1), jnp.float32)] * 2
            + [pltpu.VMEM((K1, H), jnp.float32)] * 2
            + [pltpu.VMEM((K1,), jnp.int32)] * 6
            + [pltpu.VMEM_SHARED((RPC, DA1), jnp.float32)]
            + [pltpu.SemaphoreType.DMA] * 10
        ),
    )(_c1_body)
    acc1, exbuf = c1(rec, gdst, ga)

    c2 = functools.partial(
        pl.kernel,
        out_type=jax.ShapeDtypeStruct((2, RPC, DA2), jnp.float32),
        mesh=_sc_mesh(),
        compiler_params=params,
        scratch_types=(
            [pltpu.VMEM((K2, DREC), jnp.float32)] * 2
            + [pltpu.VMEM((K2, H), jnp.float32)] * 2
            + [pltpu.VMEM((K2, DGB), jnp.float32)] * 2
            + [pltpu.VMEM((K2, DA2), jnp.float32)] * 2
            + [pltpu.VMEM((K2,), jnp.int32)] * 6
            + [pltpu.VMEM_SHARED((RPC, DA2), jnp.float32)]
            + [pltpu.SemaphoreType.DMA] * 8
        ),
    )(_c2_body)
    acc2 = c2(rec, exbuf, gb)

    a1 = jnp.concatenate([acc1[0, :HALF], acc1[1, :HALF]], axis=0)
    a2 = jnp.concatenate([acc2[0, :HALF], acc2[1, :HALF]], axis=0)
    return a1, a2


def _ln(x):
    m = x.mean(-1, keepdims=True)
    v = ((x - m) ** 2).mean(-1, keepdims=True)
    return (x - m) * lax.rsqrt(v + 1e-5)


def _epi_body(nf, a1, a2, xt, r1, r2, wo, wt1, wt2, out):
    den = a1[:, :H]
    dinv = 1.0 / jnp.maximum(den, 1e-30)
    rep1 = jnp.dot(dinv, r1[...], preferred_element_type=jnp.float32)
    rep2 = jnp.dot(dinv, r2[...], preferred_element_type=jnp.float32)
    ov = a1[:, 16:16 + CS] * rep1
    op = a2[...] * rep2 - xt[...]
    u = jnp.concatenate([ov, op], axis=-1)
    o = jnp.dot(u, wo[...], preferred_element_type=jnp.float32)
    s = _ln(nf[...] + o)
    t = jnp.dot(jax.nn.relu(jnp.dot(s, wt1[...], preferred_element_type=jnp.float32)),
                wt2[...], preferred_element_type=jnp.float32)
    out[...] = _ln(s + t)


def _epilogue(nf, a1, a2, xt, r1, r2, wo, wt1, wt2):
    grid = (N // BN_A,)
    row = lambda i: (i, 0)
    full = lambda i: (0, 0)
    return pl.pallas_call(
        _epi_body,
        grid=grid,
        in_specs=[
            pl.BlockSpec((BN_A, CS), row),
            pl.BlockSpec((BN_A, DA1), row),
            pl.BlockSpec((BN_A, DA2), row),
            pl.BlockSpec((BN_A, DVP), row),
            pl.BlockSpec((H, CS), full),
            pl.BlockSpec((H, DVP), full),
            pl.BlockSpec((CS + DVP, CS), full),
            pl.BlockSpec((CS, CS), full),
            pl.BlockSpec((CS, CS), full),
        ],
        out_specs=pl.BlockSpec((BN_A, CS), row),
        out_shape=jax.ShapeDtypeStruct((N, CS), jnp.float32),
    )(nf, a1, a2, xt, r1, r2, wo, wt1, wt2)


def kernel(node_features, edge_features, edge_index, x_ca, Wq, Wk, Wv,
           Wqp, Wkp, Wvp, Wb, Wo, Wt1, Wt2):
    eib = lax.bitcast_convert_type(
        edge_index.astype(jnp.int32).T, jnp.float32)
    xt = jnp.tile(x_ca, (1, H * PV))
    r1 = jnp.asarray(np.kron(np.eye(H, dtype=np.float32),
                             np.ones((1, CH), np.float32)))
    r2 = jnp.asarray(np.kron(np.eye(H, dtype=np.float32),
                             np.ones((1, PV * 3), np.float32)))
    gdst, ga, gb = _projections(node_features, xt, Wq, Wk, Wv, Wqp, Wkp, Wvp)
    rec = _edge_record(eib, edge_features, Wb)
    rec = jnp.pad(rec, ((0, EPAD - E), (0, 0)))
    a1, a2 = _edge_phase_sc(rec, gdst, ga, gb)
    return _epilogue(node_features, a1, a2, xt, r1, r2, Wo, Wt1, Wt2)


# speedup vs baseline: 1.6622x; 1.6622x over previous
import functools

import jax
import jax.numpy as jnp
import numpy as np
from jax import lax
from jax.experimental import pallas as pl
from jax.experimental.pallas import tpu as pltpu
from jax.experimental.pallas import tpu_sc as plsc

N = 10000
E = 320000
CS = 128
CZ = 128
H = 8
CH = 16
PQK = 4
PV = 8

DQP = H * PQK * 3
DVP = H * PV * 3
DQPP = H * CH
DDST = CS + DQPP
DGA = CS + DQPP + CS
DGB = DVP
DREC = 16
DA1 = 144
DA2 = 192

NSUB = 16
EPT = 20096
EPAD = NSUB * EPT
K1 = 32
K2 = 64
NC1 = EPT // K1
NC2 = EPT // K2
HALF = 5000
RPC = 5120
ROWS_PT = RPC // NSUB

BN_A = 1000
BN_B = 8000


def _proj_body(nf, xt, xqp, wq, wk, wv, wqp, wkp, wvp, gdst, ga, gb):
    x = nf[...]
    xq = xqp[...]
    q = jnp.dot(x, wq[...], preferred_element_type=jnp.float32)
    qp = jnp.dot(x, wqp[...], preferred_element_type=jnp.float32) + xq
    gdst[...] = jnp.concatenate([q, qp], axis=-1)
    k = jnp.dot(x, wk[...], preferred_element_type=jnp.float32)
    kp = jnp.dot(x, wkp[...], preferred_element_type=jnp.float32) + xq
    v = jnp.dot(x, wv[...], preferred_element_type=jnp.float32)
    ga[...] = jnp.concatenate([k, kp, v], axis=-1)
    gb[...] = jnp.dot(x, wvp[...], preferred_element_type=jnp.float32) + xt[...]


def _projections(nf, xt, xqp, wq, wk, wv, wqp, wkp, wvp):
    grid = (N // BN_A,)
    row = lambda i: (i, 0)
    full = lambda i: (0, 0)
    return pl.pallas_call(
        _proj_body,
        grid=grid,
        in_specs=[
            pl.BlockSpec((BN_A, CS), row),
            pl.BlockSpec((BN_A, DVP), row),
            pl.BlockSpec((BN_A, CS), row),
            pl.BlockSpec((CS, CS), full),
            pl.BlockSpec((CS, CS), full),
            pl.BlockSpec((CS, CS), full),
            pl.BlockSpec((CS, CS), full),
            pl.BlockSpec((CS, CS), full),
            pl.BlockSpec((CS, DVP), full),
        ],
        out_specs=[
            pl.BlockSpec((BN_A, DDST), row),
            pl.BlockSpec((BN_A, DGA), row),
            pl.BlockSpec((BN_A, DGB), row),
        ],
        out_shape=[
            jax.ShapeDtypeStruct((N, DDST), jnp.float32),
            jax.ShapeDtypeStruct((N, DGA), jnp.float32),
            jax.ShapeDtypeStruct((N, DGB), jnp.float32),
        ],
    )(nf, xt, xqp, wq, wk, wv, wqp, wkp, wvp)


def _rec_body(eib, ef, wb, out):
    b = jnp.dot(ef[...], wb[...], preferred_element_type=jnp.float32)
    z = jnp.zeros((BN_B, 6), jnp.float32)
    out[...] = jnp.concatenate([eib[...], z, b], axis=-1)


def _edge_record(eib, ef, wb):
    grid = (E // BN_B,)
    return pl.pallas_call(
        _rec_body,
        grid=grid,
        in_specs=[
            pl.BlockSpec((BN_B, 2), lambda i: (i, 0)),
            pl.BlockSpec((BN_B, CZ), lambda i: (i, 0)),
            pl.BlockSpec((CZ, H), lambda i: (0, 0)),
        ],
        out_specs=pl.BlockSpec((BN_B, DREC), lambda i: (i, 0)),
        out_shape=jax.ShapeDtypeStruct((E, DREC), jnp.float32),
    )(eib, ef, wb)


def _sc_mesh():
    return plsc.VectorSubcoreMesh(core_axis_name="c", subcore_axis_name="s")


def _zero_acc(ct, acc_sh, s_idx, kc):
    zero16 = jnp.zeros((16,), jnp.float32)
    width = ct.shape[1]
    for r in range(kc):
        for cc in range(width // 16):
            ct[r, pl.ds(cc * 16, 16)] = zero16
    for off in range(0, ROWS_PT, kc):
        sz = min(kc, ROWS_PT - off)
        pltpu.sync_copy(ct.at[pl.ds(0, sz)],
                        acc_sh.at[pl.ds(s_idx * ROWS_PT + off, sz)])


def _copy_out(acc_sh, out_hbm, c_idx, s_idx):
    pltpu.sync_copy(acc_sh.at[pl.ds(s_idx * ROWS_PT, ROWS_PT)],
                    out_hbm.at[c_idx, pl.ds(s_idx * ROWS_PT, ROWS_PT)])


def _c1_body(rec_hbm, gdst_hbm, ga_hbm, acc_hbm, ex_hbm,
             rec0, rec1, gd0, gd1, ga0, ga1, ct0, ct1, exv0, exv1,
             srcv0, srcv1, dstv0, dstv1, idxm0, idxm1, acc_sh,
             srec0, srec1, sgd0, sgd1, sga0, sga1, ssc0, ssc1, sex0, sex1):
    c_idx = lax.axis_index("c")
    s_idx = lax.axis_index("s")
    lo = c_idx * HALF
    iot = lax.iota(jnp.int32, 16)
    base = s_idx * EPT
    is0 = c_idx == 0

    rec = (rec0, rec1)
    gd = (gd0, gd1)
    ga = (ga0, ga1)
    ct = (ct0, ct1)
    exv = (exv0, exv1)
    srcv = (srcv0, srcv1)
    dstv = (dstv0, dstv1)
    idxm = (idxm0, idxm1)
    srec = (srec0, srec1)
    sgd = (sgd0, sgd1)
    sga = (sga0, sga1)
    ssc = (ssc0, ssc1)
    sex = (sex0, sex1)

    _zero_acc(ct0, acc_sh, s_idx, K1)
    plsc.subcore_barrier()

    def fetch_rec(h, s):
        pltpu.async_copy(rec_hbm.at[pl.ds(base + h * K1, K1)], rec[s], srec[s])

    def extract(h, s):
        e0 = base + h * K1
        for g in range(K1 // 16):
            r = g * 16 + iot
            sf = plsc.load_gather(rec[s], [r, jnp.zeros((16,), jnp.int32)])
            df = plsc.load_gather(rec[s], [r, jnp.full((16,), 1, jnp.int32)])
            sv = plsc.bitcast(sf, jnp.int32)
            dv = plsc.bitcast(df, jnp.int32)
            srcv[s][pl.ds(g * 16, 16)] = sv
            dstv[s][pl.ds(g * 16, 16)] = dv
            loc = dv - lo
            ok = (loc >= 0) & (loc < HALF) & ((iot + (e0 + g * 16)) < E)
            idxm[s][pl.ds(g * 16, 16)] = jnp.where(ok, loc, HALF)
        pltpu.async_copy(gdst_hbm.at[dstv[s]], gd[s], sgd[s])
        pltpu.async_copy(ga_hbm.at[srcv[s]], ga[s], sga[s])

    def compute_pair(h0):
        def edge_body(e, carry):
            for s in (0, 1):
                exrow = jnp.zeros((16,), jnp.float32)
                brow = rec[s][e, pl.ds(0, 16)]
                for hh in range(H):
                    qh = gd[s][e, pl.ds(hh * CH, 16)]
                    kh = ga[s][e, pl.ds(hh * CH, 16)]
                    lq = jnp.sum(qh * kh)
                    dh = (gd[s][e, pl.ds(CS + hh * CH, 16)]
                          - ga[s][e, pl.ds(CS + hh * CH, 16)])
                    pd = jnp.sum(dh * dh)
                    bh = brow[8 + hh]
                    logit = lq * 0.25 + bh - 0.1 * pd
                    exh = jnp.exp(jnp.full((16,), logit, jnp.float32))
                    vh = ga[s][e, pl.ds(CS + DQPP + hh * CH, 16)]
                    ct[s][e, pl.ds(16 + hh * CH, 16)] = exh * vh
                    exrow = jnp.where(iot == hh, exh, exrow)
                ct[s][e, pl.ds(0, 16)] = exrow
                exv[s][e, pl.ds(0, 16)] = exrow
            return carry

        lax.fori_loop(0, K1, edge_body, 0)
        for s in (0, 1):
            e0 = base + (h0 + s) * K1
            pltpu.async_copy(ct[s], acc_sh.at[idxm[s]], ssc[s], add=True)

            @pl.when(is0)
            def _(s=s, e0=e0):
                pltpu.async_copy(exv[s], ex_hbm.at[pl.ds(e0, K1)], sex[s])

    fetch_rec(0, 0)
    fetch_rec(1, 1)

    npair = NC1 // 2

    def body(p, carry):
        h0 = 2 * p
        for s in (0, 1):
            h = h0 + s

            @pl.when(p > 0)
            def _(s=s):
                pltpu.make_async_copy(ct[s], acc_sh.at[idxm[s]], ssc[s]).wait()

                @pl.when(is0)
                def _():
                    pltpu.make_async_copy(exv[s], ex_hbm.at[pl.ds(0, K1)],
                                          sex[s]).wait()

            pltpu.make_async_copy(rec_hbm.at[pl.ds(0, K1)], rec[s], srec[s]).wait()
            extract(h, s)
        for s in (0, 1):
            pltpu.make_async_copy(gdst_hbm.at[dstv[s]], gd[s], sgd[s]).wait()
            pltpu.make_async_copy(ga_hbm.at[srcv[s]], ga[s], sga[s]).wait()
        compute_pair(h0)
        for s in (0, 1):

            @pl.when(p < npair - 1)
            def _(h=h0 + s, s=s):
                fetch_rec(h + 2, s)

        return carry

    lax.fori_loop(0, npair, body, 0)
    for s in (0, 1):
        pltpu.make_async_copy(ct[s], acc_sh.at[idxm[s]], ssc[s]).wait()

        @pl.when(is0)
        def _(s=s):
            pltpu.make_async_copy(exv[s], ex_hbm.at[pl.ds(0, K1)], sex[s]).wait()

    plsc.subcore_barrier()
    _copy_out(acc_sh, acc_hbm, c_idx, s_idx)


def _c2_body(rec_hbm, ex_hbm, gb_hbm, acc_hbm,
             rec0, rec1, exv0, exv1, gb0, gb1, ct0, ct1,
             srcv0, srcv1, dstv0, dstv1, idxm0, idxm1, acc_sh,
             srec0, srec1, sev0, sev1, sgb0, sgb1, ssc0, ssc1):
    c_idx = lax.axis_index("c")
    s_idx = lax.axis_index("s")
    lo = c_idx * HALF
    iot = lax.iota(jnp.int32, 16)
    base = s_idx * EPT

    rec = (rec0, rec1)
    exv = (exv0, exv1)
    gb = (gb0, gb1)
    ct = (ct0, ct1)
    srcv = (srcv0, srcv1)
    dstv = (dstv0, dstv1)
    idxm = (idxm0, idxm1)
    srec = (srec0, srec1)
    sev = (sev0, sev1)
    sgb = (sgb0, sgb1)
    ssc = (ssc0, ssc1)

    _zero_acc(ct0, acc_sh, s_idx, K2)
    plsc.subcore_barrier()

    def fetch(h, s):
        pltpu.async_copy(rec_hbm.at[pl.ds(base + h * K2, K2)], rec[s], srec[s])
        pltpu.async_copy(ex_hbm.at[pl.ds(base + h * K2, K2)], exv[s], sev[s])

    def extract(h, s):
        e0 = base + h * K2
        for g in range(K2 // 16):
            r = g * 16 + iot
            df = plsc.load_gather(rec[s], [r, jnp.full((16,), 1, jnp.int32)])
            sf = plsc.load_gather(rec[s], [r, jnp.zeros((16,), jnp.int32)])
            dv = plsc.bitcast(df, jnp.int32)
            srcv[s][pl.ds(g * 16, 16)] = plsc.bitcast(sf, jnp.int32)
            loc = dv - lo
            ok = (loc >= 0) & (loc < HALF) & ((iot + (e0 + g * 16)) < E)
            idxm[s][pl.ds(g * 16, 16)] = jnp.where(ok, loc, HALF)
        pltpu.async_copy(gb_hbm.at[srcv[s]], gb[s], sgb[s])

    def compute_pair():
        def edge_body(e, carry):
            for s in (0, 1):
                exrow = exv[s][e, pl.ds(0, 16)]
                exs = [jnp.full((16,), exrow[hh], jnp.float32)
                       for hh in range(H)]
                for j in range(DGB // 16):
                    a = (16 * j) // 24
                    b = (16 * j + 15) // 24
                    if a == b:
                        exj = exs[a]
                    else:
                        exj = jnp.where(iot < (24 * (a + 1) - 16 * j),
                                        exs[a], exs[b])
                    ct[s][e, pl.ds(16 * j, 16)] = exj * gb[s][e, pl.ds(16 * j, 16)]
            return carry

        lax.fori_loop(0, K2, edge_body, 0)
        for s in (0, 1):
            pltpu.async_copy(ct[s], acc_sh.at[idxm[s]], ssc[s], add=True)

    fetch(0, 0)
    fetch(1, 1)
    npair = NC2 // 2

    def body(p, carry):
        h0 = 2 * p
        for s in (0, 1):
            h = h0 + s

            @pl.when(p > 0)
            def _(s=s):
                pltpu.make_async_copy(ct[s], acc_sh.at[idxm[s]], ssc[s]).wait()

            pltpu.make_async_copy(rec_hbm.at[pl.ds(0, K2)], rec[s], srec[s]).wait()
            extract(h, s)
        for s in (0, 1):
            pltpu.make_async_copy(ex_hbm.at[pl.ds(0, K2)], exv[s], sev[s]).wait()
            pltpu.make_async_copy(gb_hbm.at[srcv[s]], gb[s], sgb[s]).wait()
        compute_pair()
        for s in (0, 1):

            @pl.when(p < npair - 1)
            def _(h=h0 + s, s=s):
                fetch(h + 2, s)

        return carry

    lax.fori_loop(0, npair, body, 0)
    for s in (0, 1):
        pltpu.make_async_copy(ct[s], acc_sh.at[idxm[s]], ssc[s]).wait()
    plsc.subcore_barrier()
    _copy_out(acc_sh, acc_hbm, c_idx, s_idx)


def _edge_phase_sc(rec, gdst, ga, gb):
    params = pltpu.CompilerParams(use_tc_tiling_on_sc=False,
                                  needs_layout_passes=False)
    c1 = functools.partial(
        pl.kernel,
        out_type=[jax.ShapeDtypeStruct((2, RPC, DA1), jnp.float32),
                  jax.ShapeDtypeStruct((EPAD, 16), jnp.float32)],
        mesh=_sc_mesh(),
        compiler_params=params,
        scratch_types=(
            [pltpu.VMEM((K1, DREC), jnp.float32)] * 2
            + [pltpu.VMEM((K1, DDST), jnp.float32)] * 2
            + [pltpu.VMEM((K1, DGA), jnp.float32)] * 2
            + [pltpu.VMEM((K1, DA1), jnp.float32)] * 2
            + [pltpu.VMEM((K1, 16), jnp.float32)] * 2
            + [pltpu.VMEM((K1,), jnp.int32)] * 6
            + [pltpu.VMEM_SHARED((RPC, DA1), jnp.float32)]
            + [pltpu.SemaphoreType.DMA] * 10
        ),
    )(_c1_body)
    acc1, exbuf = c1(rec, gdst, ga)

    c2 = functools.partial(
        pl.kernel,
        out_type=jax.ShapeDtypeStruct((2, RPC, DA2), jnp.float32),
        mesh=_sc_mesh(),
        compiler_params=params,
        scratch_types=(
            [pltpu.VMEM((K2, DREC), jnp.float32)] * 2
            + [pltpu.VMEM((K2, 16), jnp.float32)] * 2
            + [pltpu.VMEM((K2, DGB), jnp.float32)] * 2
            + [pltpu.VMEM((K2, DA2), jnp.float32)] * 2
            + [pltpu.VMEM((K2,), jnp.int32)] * 6
            + [pltpu.VMEM_SHARED((RPC, DA2), jnp.float32)]
            + [pltpu.SemaphoreType.DMA] * 8
        ),
    )(_c2_body)
    acc2 = c2(rec, exbuf, gb)

    a1 = jnp.concatenate([acc1[0, :HALF], acc1[1, :HALF]], axis=0)
    a2 = jnp.concatenate([acc2[0, :HALF], acc2[1, :HALF]], axis=0)
    return a1, a2


def _ln(x):
    m = x.mean(-1, keepdims=True)
    v = ((x - m) ** 2).mean(-1, keepdims=True)
    return (x - m) * lax.rsqrt(v + 1e-5)


def _epi_body(nf, a1, a2, xt, r1, r2, wo, wt1, wt2, out):
    den = a1[:, :H]
    dinv = 1.0 / jnp.maximum(den, 1e-30)
    rep1 = jnp.dot(dinv, r1[...], preferred_element_type=jnp.float32)
    rep2 = jnp.dot(dinv, r2[...], preferred_element_type=jnp.float32)
    ov = a1[:, 16:16 + CS] * rep1
    op = a2[...] * rep2 - xt[...]
    u = jnp.concatenate([ov, op], axis=-1)
    o = jnp.dot(u, wo[...], preferred_element_type=jnp.float32)
    s = _ln(nf[...] + o)
    t = jnp.dot(jax.nn.relu(jnp.dot(s, wt1[...], preferred_element_type=jnp.float32)),
                wt2[...], preferred_element_type=jnp.float32)
    out[...] = _ln(s + t)


def _epilogue(nf, a1, a2, xt, r1, r2, wo, wt1, wt2):
    grid = (N // BN_A,)
    row = lambda i: (i, 0)
    full = lambda i: (0, 0)
    return pl.pallas_call(
        _epi_body,
        grid=grid,
        in_specs=[
            pl.BlockSpec((BN_A, CS), row),
            pl.BlockSpec((BN_A, DA1), row),
            pl.BlockSpec((BN_A, DA2), row),
            pl.BlockSpec((BN_A, DVP), row),
            pl.BlockSpec((H, CS), full),
            pl.BlockSpec((H, DVP), full),
            pl.BlockSpec((CS + DVP, CS), full),
            pl.BlockSpec((CS, CS), full),
            pl.BlockSpec((CS, CS), full),
        ],
        out_specs=pl.BlockSpec((BN_A, CS), row),
        out_shape=jax.ShapeDtypeStruct((N, CS), jnp.float32),
    )(nf, a1, a2, xt, r1, r2, wo, wt1, wt2)


def kernel(node_features, edge_features, edge_index, x_ca, Wq, Wk, Wv,
           Wqp, Wkp, Wvp, Wb, Wo, Wt1, Wt2):
    eib = lax.bitcast_convert_type(
        edge_index.astype(jnp.int32).T, jnp.float32)
    xt = jnp.tile(x_ca, (1, H * PV))
    r1 = jnp.asarray(np.kron(np.eye(H, dtype=np.float32),
                             np.ones((1, CH), np.float32)))
    r2 = jnp.asarray(np.kron(np.eye(H, dtype=np.float32),
                             np.ones((1, PV * 3), np.float32)))
    wqp_pad = jnp.pad(Wqp.reshape(CS, H, PQK * 3),
                      ((0, 0), (0, 0), (0, 4))).reshape(CS, CS)
    wkp_pad = jnp.pad(Wkp.reshape(CS, H, PQK * 3),
                      ((0, 0), (0, 0), (0, 4))).reshape(CS, CS)
    xqh = jnp.concatenate([jnp.tile(x_ca, (1, PQK)),
                           jnp.zeros((N, 4), jnp.float32)], axis=1)
    xqp = jnp.tile(xqh, (1, H))
    gdst, ga, gb = _projections(node_features, xt, xqp, Wq, Wk, Wv,
                                wqp_pad, wkp_pad, Wvp)
    rec = _edge_record(eib, edge_features, Wb)
    rec = jnp.pad(rec, ((0, EPAD - E), (0, 0)))
    a1, a2 = _edge_phase_sc(rec, gdst, ga, gb)
    return _epilogue(node_features, a1, a2, xt, r1, r2, Wo, Wt1, Wt2)


# speedup vs baseline: 1.6982x; 1.0216x over previous
import functools

import jax
import jax.numpy as jnp
import numpy as np
from jax import lax
from jax.experimental import pallas as pl
from jax.experimental.pallas import tpu as pltpu
from jax.experimental.pallas import tpu_sc as plsc

N = 10000
E = 320000
CS = 128
CZ = 128
H = 8
CH = 16
PQK = 4
PV = 8

DQP = H * PQK * 3
DVP = H * PV * 3
DQPP = H * CH
DDST = CS + DQPP
DGA = CS + DQPP + CS
DGB = DVP
DREC = 16
DA1 = 144
DA2 = 192

NSUB = 16
EPT = 20096
EPAD = NSUB * EPT
K1 = 32
K2 = 64
NC1 = EPT // K1
NC2 = EPT // K2
HALF = 5000
RPC = 5120
ROWS_PT = RPC // NSUB

BN_A = 1000
BN_B = 8000


def _proj_body(nf, xt, xqp, wq, wk, wv, wqp, wkp, wvp, gdst, ga, gb):
    x = nf[...]
    xq = xqp[...]
    q = jnp.dot(x, wq[...], preferred_element_type=jnp.float32)
    qp = jnp.dot(x, wqp[...], preferred_element_type=jnp.float32) + xq
    gdst[...] = jnp.concatenate([q, qp], axis=-1)
    k = jnp.dot(x, wk[...], preferred_element_type=jnp.float32)
    kp = jnp.dot(x, wkp[...], preferred_element_type=jnp.float32) + xq
    v = jnp.dot(x, wv[...], preferred_element_type=jnp.float32)
    ga[...] = jnp.concatenate([k, kp, v], axis=-1)
    gb[...] = jnp.dot(x, wvp[...], preferred_element_type=jnp.float32) + xt[...]


def _projections(nf, xt, xqp, wq, wk, wv, wqp, wkp, wvp):
    grid = (N // BN_A,)
    row = lambda i: (i, 0)
    full = lambda i: (0, 0)
    return pl.pallas_call(
        _proj_body,
        grid=grid,
        in_specs=[
            pl.BlockSpec((BN_A, CS), row),
            pl.BlockSpec((BN_A, DVP), row),
            pl.BlockSpec((BN_A, CS), row),
            pl.BlockSpec((CS, CS), full),
            pl.BlockSpec((CS, CS), full),
            pl.BlockSpec((CS, CS), full),
            pl.BlockSpec((CS, CS), full),
            pl.BlockSpec((CS, CS), full),
            pl.BlockSpec((CS, DVP), full),
        ],
        out_specs=[
            pl.BlockSpec((BN_A, DDST), row),
            pl.BlockSpec((BN_A, DGA), row),
            pl.BlockSpec((BN_A, DGB), row),
        ],
        out_shape=[
            jax.ShapeDtypeStruct((N, DDST), jnp.float32),
            jax.ShapeDtypeStruct((N, DGA), jnp.float32),
            jax.ShapeDtypeStruct((N, DGB), jnp.float32),
        ],
    )(nf, xt, xqp, wq, wk, wv, wqp, wkp, wvp)


def _rec_body(eib, ef, wb, out):
    b = jnp.dot(ef[...], wb[...], preferred_element_type=jnp.float32)
    z = jnp.zeros((BN_B, 6), jnp.float32)
    out[...] = jnp.concatenate([eib[...], z, b], axis=-1)


def _edge_record(eib, ef, wb):
    grid = (E // BN_B,)
    return pl.pallas_call(
        _rec_body,
        grid=grid,
        in_specs=[
            pl.BlockSpec((BN_B, 2), lambda i: (i, 0)),
            pl.BlockSpec((BN_B, CZ), lambda i: (i, 0)),
            pl.BlockSpec((CZ, H), lambda i: (0, 0)),
        ],
        out_specs=pl.BlockSpec((BN_B, DREC), lambda i: (i, 0)),
        out_shape=jax.ShapeDtypeStruct((E, DREC), jnp.float32),
    )(eib, ef, wb)


def _sc_mesh():
    return plsc.VectorSubcoreMesh(core_axis_name="c", subcore_axis_name="s")


def _zero_acc(ct, acc_sh, s_idx, kc):
    zero16 = jnp.zeros((16,), jnp.float32)
    width = ct.shape[1]
    for r in range(kc):
        for cc in range(width // 16):
            ct[r, pl.ds(cc * 16, 16)] = zero16
    for off in range(0, ROWS_PT, kc):
        sz = min(kc, ROWS_PT - off)
        pltpu.sync_copy(ct.at[pl.ds(0, sz)],
                        acc_sh.at[pl.ds(s_idx * ROWS_PT + off, sz)])


def _copy_out(acc_sh, out_hbm, c_idx, s_idx):
    pltpu.sync_copy(acc_sh.at[pl.ds(s_idx * ROWS_PT, ROWS_PT)],
                    out_hbm.at[c_idx, pl.ds(s_idx * ROWS_PT, ROWS_PT)])


def _c1_body(rec_hbm, gdst_hbm, ga_hbm, acc_hbm, ex_hbm,
             rec0, rec1, gd0, gd1, ga0, ga1, ct0, ct1, exv0, exv1,
             srcv0, srcv1, dstv0, dstv1, idxm0, idxm1, acc_sh,
             srec0, srec1, sgd0, sgd1, sga0, sga1, ssc0, ssc1, sex0, sex1):
    c_idx = lax.axis_index("c")
    s_idx = lax.axis_index("s")
    lo = c_idx * HALF
    iot = lax.iota(jnp.int32, 16)
    base = s_idx * EPT
    is0 = c_idx == 0

    rec = (rec0, rec1)
    gd = (gd0, gd1)
    ga = (ga0, ga1)
    ct = (ct0, ct1)
    exv = (exv0, exv1)
    srcv = (srcv0, srcv1)
    dstv = (dstv0, dstv1)
    idxm = (idxm0, idxm1)
    srec = (srec0, srec1)
    sgd = (sgd0, sgd1)
    sga = (sga0, sga1)
    ssc = (ssc0, ssc1)
    sex = (sex0, sex1)

    _zero_acc(ct0, acc_sh, s_idx, K1)
    plsc.subcore_barrier()

    def fetch_rec(h, s):
        pltpu.async_copy(rec_hbm.at[pl.ds(base + h * K1, K1)], rec[s], srec[s])

    def extract(h, s):
        e0 = base + h * K1
        for g in range(K1 // 16):
            r = g * 16 + iot
            sf = plsc.load_gather(rec[s], [r, jnp.zeros((16,), jnp.int32)])
            df = plsc.load_gather(rec[s], [r, jnp.full((16,), 1, jnp.int32)])
            sv = plsc.bitcast(sf, jnp.int32)
            dv = plsc.bitcast(df, jnp.int32)
            srcv[s][pl.ds(g * 16, 16)] = sv
            dstv[s][pl.ds(g * 16, 16)] = dv
            loc = dv - lo
            ok = (loc >= 0) & (loc < HALF) & ((iot + (e0 + g * 16)) < E)
            idxm[s][pl.ds(g * 16, 16)] = jnp.where(ok, loc, HALF)
        pltpu.async_copy(gdst_hbm.at[dstv[s]], gd[s], sgd[s])
        pltpu.async_copy(ga_hbm.at[srcv[s]], ga[s], sga[s])

    def compute_pair(h0):
        @plsc.parallel_loop(0, K1, unroll=2)
        def _(e):
            for s in (0, 1):
                exrow = jnp.zeros((16,), jnp.float32)
                brow = rec[s][e, pl.ds(0, 16)]
                for hh in range(H):
                    qh = gd[s][e, pl.ds(hh * CH, 16)]
                    kh = ga[s][e, pl.ds(hh * CH, 16)]
                    dh = (gd[s][e, pl.ds(CS + hh * CH, 16)]
                          - ga[s][e, pl.ds(CS + hh * CH, 16)])
                    m = qh * kh * 0.25 - dh * dh * 0.1
                    logit = jnp.sum(m) + brow[8 + hh]
                    exh = jnp.exp(jnp.full((16,), logit, jnp.float32))
                    vh = ga[s][e, pl.ds(CS + DQPP + hh * CH, 16)]
                    ct[s][e, pl.ds(16 + hh * CH, 16)] = exh * vh
                    exrow = jnp.where(iot == hh, exh, exrow)
                ct[s][e, pl.ds(0, 16)] = exrow
                exv[s][e, pl.ds(0, 16)] = exrow
        for s in (0, 1):
            e0 = base + (h0 + s) * K1
            pltpu.async_copy(ct[s], acc_sh.at[idxm[s]], ssc[s], add=True)

            @pl.when(is0)
            def _(s=s, e0=e0):
                pltpu.async_copy(exv[s], ex_hbm.at[pl.ds(e0, K1)], sex[s])

    fetch_rec(0, 0)
    fetch_rec(1, 1)

    npair = NC1 // 2

    def body(p, carry):
        h0 = 2 * p
        for s in (0, 1):
            h = h0 + s

            @pl.when(p > 0)
            def _(s=s):
                pltpu.make_async_copy(ct[s], acc_sh.at[idxm[s]], ssc[s]).wait()

                @pl.when(is0)
                def _():
                    pltpu.make_async_copy(exv[s], ex_hbm.at[pl.ds(0, K1)],
                                          sex[s]).wait()

            pltpu.make_async_copy(rec_hbm.at[pl.ds(0, K1)], rec[s], srec[s]).wait()
            extract(h, s)
        for s in (0, 1):
            pltpu.make_async_copy(gdst_hbm.at[dstv[s]], gd[s], sgd[s]).wait()
            pltpu.make_async_copy(ga_hbm.at[srcv[s]], ga[s], sga[s]).wait()
        compute_pair(h0)
        for s in (0, 1):

            @pl.when(p < npair - 1)
            def _(h=h0 + s, s=s):
                fetch_rec(h + 2, s)

        return carry

    lax.fori_loop(0, npair, body, 0)
    for s in (0, 1):
        pltpu.make_async_copy(ct[s], acc_sh.at[idxm[s]], ssc[s]).wait()

        @pl.when(is0)
        def _(s=s):
            pltpu.make_async_copy(exv[s], ex_hbm.at[pl.ds(0, K1)], sex[s]).wait()

    plsc.subcore_barrier()
    _copy_out(acc_sh, acc_hbm, c_idx, s_idx)


def _c2_body(rec_hbm, ex_hbm, gb_hbm, acc_hbm,
             rec0, rec1, exv0, exv1, gb0, gb1, ct0, ct1,
             srcv0, srcv1, dstv0, dstv1, idxm0, idxm1, acc_sh,
             srec0, srec1, sev0, sev1, sgb0, sgb1, ssc0, ssc1):
    c_idx = lax.axis_index("c")
    s_idx = lax.axis_index("s")
    lo = c_idx * HALF
    iot = lax.iota(jnp.int32, 16)
    base = s_idx * EPT

    rec = (rec0, rec1)
    exv = (exv0, exv1)
    gb = (gb0, gb1)
    ct = (ct0, ct1)
    srcv = (srcv0, srcv1)
    dstv = (dstv0, dstv1)
    idxm = (idxm0, idxm1)
    srec = (srec0, srec1)
    sev = (sev0, sev1)
    sgb = (sgb0, sgb1)
    ssc = (ssc0, ssc1)

    _zero_acc(ct0, acc_sh, s_idx, K2)
    plsc.subcore_barrier()

    def fetch(h, s):
        pltpu.async_copy(rec_hbm.at[pl.ds(base + h * K2, K2)], rec[s], srec[s])
        pltpu.async_copy(ex_hbm.at[pl.ds(base + h * K2, K2)], exv[s], sev[s])

    def extract(h, s):
        e0 = base + h * K2
        for g in range(K2 // 16):
            r = g * 16 + iot
            df = plsc.load_gather(rec[s], [r, jnp.full((16,), 1, jnp.int32)])
            sf = plsc.load_gather(rec[s], [r, jnp.zeros((16,), jnp.int32)])
            dv = plsc.bitcast(df, jnp.int32)
            srcv[s][pl.ds(g * 16, 16)] = plsc.bitcast(sf, jnp.int32)
            loc = dv - lo
            ok = (loc >= 0) & (loc < HALF) & ((iot + (e0 + g * 16)) < E)
            idxm[s][pl.ds(g * 16, 16)] = jnp.where(ok, loc, HALF)
        pltpu.async_copy(gb_hbm.at[srcv[s]], gb[s], sgb[s])

    def compute_pair():
        @plsc.parallel_loop(0, K2, unroll=2)
        def _(e):
            for s in (0, 1):
                exrow = exv[s][e, pl.ds(0, 16)]
                exs = [jnp.full((16,), exrow[hh], jnp.float32)
                       for hh in range(H)]
                for j in range(DGB // 16):
                    a = (16 * j) // 24
                    b = (16 * j + 15) // 24
                    if a == b:
                        exj = exs[a]
                    else:
                        exj = jnp.where(iot < (24 * (a + 1) - 16 * j),
                                        exs[a], exs[b])
                    ct[s][e, pl.ds(16 * j, 16)] = exj * gb[s][e, pl.ds(16 * j, 16)]
        for s in (0, 1):
            pltpu.async_copy(ct[s], acc_sh.at[idxm[s]], ssc[s], add=True)

    fetch(0, 0)
    fetch(1, 1)
    npair = NC2 // 2

    def body(p, carry):
        h0 = 2 * p
        for s in (0, 1):
            h = h0 + s

            @pl.when(p > 0)
            def _(s=s):
                pltpu.make_async_copy(ct[s], acc_sh.at[idxm[s]], ssc[s]).wait()

            pltpu.make_async_copy(rec_hbm.at[pl.ds(0, K2)], rec[s], srec[s]).wait()
            extract(h, s)
        for s in (0, 1):
            pltpu.make_async_copy(ex_hbm.at[pl.ds(0, K2)], exv[s], sev[s]).wait()
            pltpu.make_async_copy(gb_hbm.at[srcv[s]], gb[s], sgb[s]).wait()
        compute_pair()
        for s in (0, 1):

            @pl.when(p < npair - 1)
            def _(h=h0 + s, s=s):
                fetch(h + 2, s)

        return carry

    lax.fori_loop(0, npair, body, 0)
    for s in (0, 1):
        pltpu.make_async_copy(ct[s], acc_sh.at[idxm[s]], ssc[s]).wait()
    plsc.subcore_barrier()
    _copy_out(acc_sh, acc_hbm, c_idx, s_idx)


def _edge_phase_sc(rec, gdst, ga, gb):
    params = pltpu.CompilerParams(use_tc_tiling_on_sc=False,
                                  needs_layout_passes=False)
    c1 = functools.partial(
        pl.kernel,
        out_type=[jax.ShapeDtypeStruct((2, RPC, DA1), jnp.float32),
                  jax.ShapeDtypeStruct((EPAD, 16), jnp.float32)],
        mesh=_sc_mesh(),
        compiler_params=params,
        scratch_types=(
            [pltpu.VMEM((K1, DREC), jnp.float32)] * 2
            + [pltpu.VMEM((K1, DDST), jnp.float32)] * 2
            + [pltpu.VMEM((K1, DGA), jnp.float32)] * 2
            + [pltpu.VMEM((K1, DA1), jnp.float32)] * 2
            + [pltpu.VMEM((K1, 16), jnp.float32)] * 2
            + [pltpu.VMEM((K1,), jnp.int32)] * 6
            + [pltpu.VMEM_SHARED((RPC, DA1), jnp.float32)]
            + [pltpu.SemaphoreType.DMA] * 10
        ),
    )(_c1_body)
    acc1, exbuf = c1(rec, gdst, ga)

    c2 = functools.partial(
        pl.kernel,
        out_type=jax.ShapeDtypeStruct((2, RPC, DA2), jnp.float32),
        mesh=_sc_mesh(),
        compiler_params=params,
        scratch_types=(
            [pltpu.VMEM((K2, DREC), jnp.float32)] * 2
            + [pltpu.VMEM((K2, 16), jnp.float32)] * 2
            + [pltpu.VMEM((K2, DGB), jnp.float32)] * 2
            + [pltpu.VMEM((K2, DA2), jnp.float32)] * 2
            + [pltpu.VMEM((K2,), jnp.int32)] * 6
            + [pltpu.VMEM_SHARED((RPC, DA2), jnp.float32)]
            + [pltpu.SemaphoreType.DMA] * 8
        ),
    )(_c2_body)
    acc2 = c2(rec, exbuf, gb)

    a1 = jnp.concatenate([acc1[0, :HALF], acc1[1, :HALF]], axis=0)
    a2 = jnp.concatenate([acc2[0, :HALF], acc2[1, :HALF]], axis=0)
    return a1, a2


def _ln(x):
    m = x.mean(-1, keepdims=True)
    v = ((x - m) ** 2).mean(-1, keepdims=True)
    return (x - m) * lax.rsqrt(v + 1e-5)


def _epi_body(nf, a1, a2, xt, r1, r2, wo, wt1, wt2, out):
    den = a1[:, :H]
    dinv = 1.0 / jnp.maximum(den, 1e-30)
    rep1 = jnp.dot(dinv, r1[...], preferred_element_type=jnp.float32)
    rep2 = jnp.dot(dinv, r2[...], preferred_element_type=jnp.float32)
    ov = a1[:, 16:16 + CS] * rep1
    op = a2[...] * rep2 - xt[...]
    u = jnp.concatenate([ov, op], axis=-1)
    o = jnp.dot(u, wo[...], preferred_element_type=jnp.float32)
    s = _ln(nf[...] + o)
    t = jnp.dot(jax.nn.relu(jnp.dot(s, wt1[...], preferred_element_type=jnp.float32)),
                wt2[...], preferred_element_type=jnp.float32)
    out[...] = _ln(s + t)


def _epilogue(nf, a1, a2, xt, r1, r2, wo, wt1, wt2):
    grid = (N // BN_A,)
    row = lambda i: (i, 0)
    full = lambda i: (0, 0)
    return pl.pallas_call(
        _epi_body,
        grid=grid,
        in_specs=[
            pl.BlockSpec((BN_A, CS), row),
            pl.BlockSpec((BN_A, DA1), row),
            pl.BlockSpec((BN_A, DA2), row),
            pl.BlockSpec((BN_A, DVP), row),
            pl.BlockSpec((H, CS), full),
            pl.BlockSpec((H, DVP), full),
            pl.BlockSpec((CS + DVP, CS), full),
            pl.BlockSpec((CS, CS), full),
            pl.BlockSpec((CS, CS), full),
        ],
        out_specs=pl.BlockSpec((BN_A, CS), row),
        out_shape=jax.ShapeDtypeStruct((N, CS), jnp.float32),
    )(nf, a1, a2, xt, r1, r2, wo, wt1, wt2)


def kernel(node_features, edge_features, edge_index, x_ca, Wq, Wk, Wv,
           Wqp, Wkp, Wvp, Wb, Wo, Wt1, Wt2):
    eib = lax.bitcast_convert_type(
        edge_index.astype(jnp.int32).T, jnp.float32)
    xt = jnp.tile(x_ca, (1, H * PV))
    r1 = jnp.asarray(np.kron(np.eye(H, dtype=np.float32),
                             np.ones((1, CH), np.float32)))
    r2 = jnp.asarray(np.kron(np.eye(H, dtype=np.float32),
                             np.ones((1, PV * 3), np.float32)))
    wqp_pad = jnp.pad(Wqp.reshape(CS, H, PQK * 3),
                      ((0, 0), (0, 0), (0, 4))).reshape(CS, CS)
    wkp_pad = jnp.pad(Wkp.reshape(CS, H, PQK * 3),
                      ((0, 0), (0, 0), (0, 4))).reshape(CS, CS)
    xqh = jnp.concatenate([jnp.tile(x_ca, (1, PQK)),
                           jnp.zeros((N, 4), jnp.float32)], axis=1)
    xqp = jnp.tile(xqh, (1, H))
    gdst, ga, gb = _projections(node_features, xt, xqp, Wq, Wk, Wv,
                                wqp_pad, wkp_pad, Wvp)
    rec = _edge_record(eib, edge_features, Wb)
    rec = jnp.pad(rec, ((0, EPAD - E), (0, 0)))
    a1, a2 = _edge_phase_sc(rec, gdst, ga, gb)
    return _epilogue(node_features, a1, a2, xt, r1, r2, Wo, Wt1, Wt2)


# speedup vs baseline: 2.0544x; 1.2098x over previous
import functools

import jax
import jax.numpy as jnp
import numpy as np
from jax import lax
from jax.experimental import pallas as pl
from jax.experimental.pallas import tpu as pltpu
from jax.experimental.pallas import tpu_sc as plsc

N = 10000
E = 320000
CS = 128
CZ = 128
H = 8
CH = 16
PQK = 4
PV = 8

DQP = H * PQK * 3
DVP = H * PV * 3
DQPP = H * CH
DDST = CS + DQPP
DGA = CS + DQPP + CS
DGB = DVP
DREC = 16
DA1 = 144
DA2 = 192

NSUB = 16
EPT = 20096
EPAD = NSUB * EPT
K1 = 32
K2 = 64
NC1 = EPT // K1
NC2 = EPT // K2
HALF = 5000
RPC = 5120
ROWS_PT = RPC // NSUB

BN_A = 1000
BN_B = 8000


def _proj_body(nf, xt, xqp, wq, wk, wv, wqp, wkp, wvp, gdst, ga, gb):
    x = nf[...]
    xq = xqp[...]
    q = jnp.dot(x, wq[...], preferred_element_type=jnp.float32)
    qp = jnp.dot(x, wqp[...], preferred_element_type=jnp.float32) + xq
    gdst[...] = jnp.concatenate([q, qp], axis=-1)
    k = jnp.dot(x, wk[...], preferred_element_type=jnp.float32)
    kp = jnp.dot(x, wkp[...], preferred_element_type=jnp.float32) + xq
    v = jnp.dot(x, wv[...], preferred_element_type=jnp.float32)
    ga[...] = jnp.concatenate([k, kp, v], axis=-1)
    gb[...] = jnp.dot(x, wvp[...], preferred_element_type=jnp.float32) + xt[...]


def _projections(nf, xt, xqp, wq, wk, wv, wqp, wkp, wvp):
    grid = (N // BN_A,)
    row = lambda i: (i, 0)
    full = lambda i: (0, 0)
    return pl.pallas_call(
        _proj_body,
        grid=grid,
        in_specs=[
            pl.BlockSpec((BN_A, CS), row),
            pl.BlockSpec((BN_A, DVP), row),
            pl.BlockSpec((BN_A, CS), row),
            pl.BlockSpec((CS, CS), full),
            pl.BlockSpec((CS, CS), full),
            pl.BlockSpec((CS, CS), full),
            pl.BlockSpec((CS, CS), full),
            pl.BlockSpec((CS, CS), full),
            pl.BlockSpec((CS, DVP), full),
        ],
        out_specs=[
            pl.BlockSpec((BN_A, DDST), row),
            pl.BlockSpec((BN_A, DGA), row),
            pl.BlockSpec((BN_A, DGB), row),
        ],
        out_shape=[
            jax.ShapeDtypeStruct((N, DDST), jnp.float32),
            jax.ShapeDtypeStruct((N, DGA), jnp.float32),
            jax.ShapeDtypeStruct((N, DGB), jnp.float32),
        ],
    )(nf, xt, xqp, wq, wk, wv, wqp, wkp, wvp)


def _rec_body(eib, ef, wb, out):
    b = jnp.dot(ef[...], wb[...], preferred_element_type=jnp.float32)
    z = jnp.zeros((BN_B, 6), jnp.float32)
    out[...] = jnp.concatenate([eib[...], z, b], axis=-1)


def _edge_record(eib, ef, wb):
    grid = (E // BN_B,)
    return pl.pallas_call(
        _rec_body,
        grid=grid,
        in_specs=[
            pl.BlockSpec((BN_B, 2), lambda i: (i, 0)),
            pl.BlockSpec((BN_B, CZ), lambda i: (i, 0)),
            pl.BlockSpec((CZ, H), lambda i: (0, 0)),
        ],
        out_specs=pl.BlockSpec((BN_B, DREC), lambda i: (i, 0)),
        out_shape=jax.ShapeDtypeStruct((E, DREC), jnp.float32),
    )(eib, ef, wb)


def _sc_mesh():
    return plsc.VectorSubcoreMesh(core_axis_name="c", subcore_axis_name="s")


def _zero_acc(ct, acc_sh, s_idx, kc):
    zero16 = jnp.zeros((16,), jnp.float32)
    width = ct.shape[1]
    for r in range(kc):
        for cc in range(width // 16):
            ct[r, pl.ds(cc * 16, 16)] = zero16
    for off in range(0, ROWS_PT, kc):
        sz = min(kc, ROWS_PT - off)
        pltpu.sync_copy(ct.at[pl.ds(0, sz)],
                        acc_sh.at[pl.ds(s_idx * ROWS_PT + off, sz)])


def _copy_out(acc_sh, out_hbm, c_idx, s_idx):
    pltpu.sync_copy(acc_sh.at[pl.ds(s_idx * ROWS_PT, ROWS_PT)],
                    out_hbm.at[c_idx, pl.ds(s_idx * ROWS_PT, ROWS_PT)])


def _c1_body(rec_hbm, gdst_hbm, ga_hbm, acc_hbm, ex_hbm,
             rec0, rec1, gd0, gd1, ga0, ga1, ct0, ct1, exv0, exv1,
             srcv0, srcv1, dstv0, dstv1, idxm0, idxm1, acc_sh,
             srec0, srec1, sgd0, sgd1, sga0, sga1, ssc0, ssc1, sex0, sex1):
    c_idx = lax.axis_index("c")
    s_idx = lax.axis_index("s")
    lo = c_idx * HALF
    iot = lax.iota(jnp.int32, 16)
    base = s_idx * EPT
    is0 = c_idx == 0

    rec = (rec0, rec1)
    gd = (gd0, gd1)
    ga = (ga0, ga1)
    ct = (ct0, ct1)
    exv = (exv0, exv1)
    srcv = (srcv0, srcv1)
    dstv = (dstv0, dstv1)
    idxm = (idxm0, idxm1)
    srec = (srec0, srec1)
    sgd = (sgd0, sgd1)
    sga = (sga0, sga1)
    ssc = (ssc0, ssc1)
    sex = (sex0, sex1)

    _zero_acc(ct0, acc_sh, s_idx, K1)
    plsc.subcore_barrier()

    def fetch_rec(h, s):
        pltpu.async_copy(rec_hbm.at[pl.ds(base + h * K1, K1)], rec[s], srec[s])

    def extract(h, s):
        e0 = base + h * K1
        for g in range(K1 // 16):
            r = g * 16 + iot
            sf = plsc.load_gather(rec[s], [r, jnp.zeros((16,), jnp.int32)])
            df = plsc.load_gather(rec[s], [r, jnp.full((16,), 1, jnp.int32)])
            sv = plsc.bitcast(sf, jnp.int32)
            dv = plsc.bitcast(df, jnp.int32)
            srcv[s][pl.ds(g * 16, 16)] = sv
            dstv[s][pl.ds(g * 16, 16)] = dv
            loc = dv - lo
            ok = (loc >= 0) & (loc < HALF) & ((iot + (e0 + g * 16)) < E)
            idxm[s][pl.ds(g * 16, 16)] = jnp.where(ok, loc, HALF)
        pltpu.async_copy(gdst_hbm.at[dstv[s]], gd[s], sgd[s])
        pltpu.async_copy(ga_hbm.at[srcv[s]], ga[s], sga[s])

    def compute_pair(h0):
        perms = [jnp.bitwise_xor(iot, k) for k in (8, 4, 2, 1)]

        def bsum(x):
            for p in perms:
                x = x + x.at[p].get(mode="promise_in_bounds")
            return x

        @plsc.parallel_loop(0, K1, unroll=2)
        def _(e):
            for s in (0, 1):
                exrow = jnp.zeros((16,), jnp.float32)
                brow = rec[s][e, pl.ds(0, 16)]
                for hh in range(H):
                    qh = gd[s][e, pl.ds(hh * CH, 16)]
                    kh = ga[s][e, pl.ds(hh * CH, 16)]
                    dh = (gd[s][e, pl.ds(CS + hh * CH, 16)]
                          - ga[s][e, pl.ds(CS + hh * CH, 16)])
                    m = qh * kh * 0.25 - dh * dh * 0.1
                    bsp = brow.at[jnp.full((16,), 8 + hh, jnp.int32)].get(
                        mode="promise_in_bounds")
                    exh = jnp.exp(bsum(m) + bsp)
                    vh = ga[s][e, pl.ds(CS + DQPP + hh * CH, 16)]
                    ct[s][e, pl.ds(16 + hh * CH, 16)] = exh * vh
                    exrow = jnp.where(iot == hh, exh, exrow)
                ct[s][e, pl.ds(0, 16)] = exrow
                exv[s][e, pl.ds(0, 16)] = exrow
        for s in (0, 1):
            e0 = base + (h0 + s) * K1
            pltpu.async_copy(ct[s], acc_sh.at[idxm[s]], ssc[s], add=True)

            @pl.when(is0)
            def _(s=s, e0=e0):
                pltpu.async_copy(exv[s], ex_hbm.at[pl.ds(e0, K1)], sex[s])

    fetch_rec(0, 0)
    fetch_rec(1, 1)

    npair = NC1 // 2

    def body(p, carry):
        h0 = 2 * p
        for s in (0, 1):
            h = h0 + s

            @pl.when(p > 0)
            def _(s=s):
                pltpu.make_async_copy(ct[s], acc_sh.at[idxm[s]], ssc[s]).wait()

                @pl.when(is0)
                def _():
                    pltpu.make_async_copy(exv[s], ex_hbm.at[pl.ds(0, K1)],
                                          sex[s]).wait()

            pltpu.make_async_copy(rec_hbm.at[pl.ds(0, K1)], rec[s], srec[s]).wait()
            extract(h, s)
        for s in (0, 1):
            pltpu.make_async_copy(gdst_hbm.at[dstv[s]], gd[s], sgd[s]).wait()
            pltpu.make_async_copy(ga_hbm.at[srcv[s]], ga[s], sga[s]).wait()
        compute_pair(h0)
        for s in (0, 1):

            @pl.when(p < npair - 1)
            def _(h=h0 + s, s=s):
                fetch_rec(h + 2, s)

        return carry

    lax.fori_loop(0, npair, body, 0)
    for s in (0, 1):
        pltpu.make_async_copy(ct[s], acc_sh.at[idxm[s]], ssc[s]).wait()

        @pl.when(is0)
        def _(s=s):
            pltpu.make_async_copy(exv[s], ex_hbm.at[pl.ds(0, K1)], sex[s]).wait()

    plsc.subcore_barrier()
    _copy_out(acc_sh, acc_hbm, c_idx, s_idx)


def _c2_body(rec_hbm, ex_hbm, gb_hbm, acc_hbm,
             rec0, rec1, exv0, exv1, gb0, gb1, ct0, ct1,
             srcv0, srcv1, dstv0, dstv1, idxm0, idxm1, acc_sh,
             srec0, srec1, sev0, sev1, sgb0, sgb1, ssc0, ssc1):
    c_idx = lax.axis_index("c")
    s_idx = lax.axis_index("s")
    lo = c_idx * HALF
    iot = lax.iota(jnp.int32, 16)
    base = s_idx * EPT

    rec = (rec0, rec1)
    exv = (exv0, exv1)
    gb = (gb0, gb1)
    ct = (ct0, ct1)
    srcv = (srcv0, srcv1)
    dstv = (dstv0, dstv1)
    idxm = (idxm0, idxm1)
    srec = (srec0, srec1)
    sev = (sev0, sev1)
    sgb = (sgb0, sgb1)
    ssc = (ssc0, ssc1)

    _zero_acc(ct0, acc_sh, s_idx, K2)
    plsc.subcore_barrier()

    def fetch(h, s):
        pltpu.async_copy(rec_hbm.at[pl.ds(base + h * K2, K2)], rec[s], srec[s])
        pltpu.async_copy(ex_hbm.at[pl.ds(base + h * K2, K2)], exv[s], sev[s])

    def extract(h, s):
        e0 = base + h * K2
        for g in range(K2 // 16):
            r = g * 16 + iot
            df = plsc.load_gather(rec[s], [r, jnp.full((16,), 1, jnp.int32)])
            sf = plsc.load_gather(rec[s], [r, jnp.zeros((16,), jnp.int32)])
            dv = plsc.bitcast(df, jnp.int32)
            srcv[s][pl.ds(g * 16, 16)] = plsc.bitcast(sf, jnp.int32)
            loc = dv - lo
            ok = (loc >= 0) & (loc < HALF) & ((iot + (e0 + g * 16)) < E)
            idxm[s][pl.ds(g * 16, 16)] = jnp.where(ok, loc, HALF)
        pltpu.async_copy(gb_hbm.at[srcv[s]], gb[s], sgb[s])

    def compute_pair():
        @plsc.parallel_loop(0, K2, unroll=2)
        def _(e):
            for s in (0, 1):
                exrow = exv[s][e, pl.ds(0, 16)]
                exs = [jnp.full((16,), exrow[hh], jnp.float32)
                       for hh in range(H)]
                for j in range(DGB // 16):
                    a = (16 * j) // 24
                    b = (16 * j + 15) // 24
                    if a == b:
                        exj = exs[a]
                    else:
                        exj = jnp.where(iot < (24 * (a + 1) - 16 * j),
                                        exs[a], exs[b])
                    ct[s][e, pl.ds(16 * j, 16)] = exj * gb[s][e, pl.ds(16 * j, 16)]
        for s in (0, 1):
            pltpu.async_copy(ct[s], acc_sh.at[idxm[s]], ssc[s], add=True)

    fetch(0, 0)
    fetch(1, 1)
    npair = NC2 // 2

    def body(p, carry):
        h0 = 2 * p
        for s in (0, 1):
            h = h0 + s

            @pl.when(p > 0)
            def _(s=s):
                pltpu.make_async_copy(ct[s], acc_sh.at[idxm[s]], ssc[s]).wait()

            pltpu.make_async_copy(rec_hbm.at[pl.ds(0, K2)], rec[s], srec[s]).wait()
            extract(h, s)
        for s in (0, 1):
            pltpu.make_async_copy(ex_hbm.at[pl.ds(0, K2)], exv[s], sev[s]).wait()
            pltpu.make_async_copy(gb_hbm.at[srcv[s]], gb[s], sgb[s]).wait()
        compute_pair()
        for s in (0, 1):

            @pl.when(p < npair - 1)
            def _(h=h0 + s, s=s):
                fetch(h + 2, s)

        return carry

    lax.fori_loop(0, npair, body, 0)
    for s in (0, 1):
        pltpu.make_async_copy(ct[s], acc_sh.at[idxm[s]], ssc[s]).wait()
    plsc.subcore_barrier()
    _copy_out(acc_sh, acc_hbm, c_idx, s_idx)


def _edge_phase_sc(rec, gdst, ga, gb):
    params = pltpu.CompilerParams(use_tc_tiling_on_sc=False,
                                  needs_layout_passes=False)
    c1 = functools.partial(
        pl.kernel,
        out_type=[jax.ShapeDtypeStruct((2, RPC, DA1), jnp.float32),
                  jax.ShapeDtypeStruct((EPAD, 16), jnp.float32)],
        mesh=_sc_mesh(),
        compiler_params=params,
        scratch_types=(
            [pltpu.VMEM((K1, DREC), jnp.float32)] * 2
            + [pltpu.VMEM((K1, DDST), jnp.float32)] * 2
            + [pltpu.VMEM((K1, DGA), jnp.float32)] * 2
            + [pltpu.VMEM((K1, DA1), jnp.float32)] * 2
            + [pltpu.VMEM((K1, 16), jnp.float32)] * 2
            + [pltpu.VMEM((K1,), jnp.int32)] * 6
            + [pltpu.VMEM_SHARED((RPC, DA1), jnp.float32)]
            + [pltpu.SemaphoreType.DMA] * 10
        ),
    )(_c1_body)
    acc1, exbuf = c1(rec, gdst, ga)

    c2 = functools.partial(
        pl.kernel,
        out_type=jax.ShapeDtypeStruct((2, RPC, DA2), jnp.float32),
        mesh=_sc_mesh(),
        compiler_params=params,
        scratch_types=(
            [pltpu.VMEM((K2, DREC), jnp.float32)] * 2
            + [pltpu.VMEM((K2, 16), jnp.float32)] * 2
            + [pltpu.VMEM((K2, DGB), jnp.float32)] * 2
            + [pltpu.VMEM((K2, DA2), jnp.float32)] * 2
            + [pltpu.VMEM((K2,), jnp.int32)] * 6
            + [pltpu.VMEM_SHARED((RPC, DA2), jnp.float32)]
            + [pltpu.SemaphoreType.DMA] * 8
        ),
    )(_c2_body)
    acc2 = c2(rec, exbuf, gb)

    a1 = jnp.concatenate([acc1[0, :HALF], acc1[1, :HALF]], axis=0)
    a2 = jnp.concatenate([acc2[0, :HALF], acc2[1, :HALF]], axis=0)
    return a1, a2


def _ln(x):
    m = x.mean(-1, keepdims=True)
    v = ((x - m) ** 2).mean(-1, keepdims=True)
    return (x - m) * lax.rsqrt(v + 1e-5)


def _epi_body(nf, a1, a2, xt, r1, r2, wo, wt1, wt2, out):
    den = a1[:, :H]
    dinv = 1.0 / jnp.maximum(den, 1e-30)
    rep1 = jnp.dot(dinv, r1[...], preferred_element_type=jnp.float32)
    rep2 = jnp.dot(dinv, r2[...], preferred_element_type=jnp.float32)
    ov = a1[:, 16:16 + CS] * rep1
    op = a2[...] * rep2 - xt[...]
    u = jnp.concatenate([ov, op], axis=-1)
    o = jnp.dot(u, wo[...], preferred_element_type=jnp.float32)
    s = _ln(nf[...] + o)
    t = jnp.dot(jax.nn.relu(jnp.dot(s, wt1[...], preferred_element_type=jnp.float32)),
                wt2[...], preferred_element_type=jnp.float32)
    out[...] = _ln(s + t)


def _epilogue(nf, a1, a2, xt, r1, r2, wo, wt1, wt2):
    grid = (N // BN_A,)
    row = lambda i: (i, 0)
    full = lambda i: (0, 0)
    return pl.pallas_call(
        _epi_body,
        grid=grid,
        in_specs=[
            pl.BlockSpec((BN_A, CS), row),
            pl.BlockSpec((BN_A, DA1), row),
            pl.BlockSpec((BN_A, DA2), row),
            pl.BlockSpec((BN_A, DVP), row),
            pl.BlockSpec((H, CS), full),
            pl.BlockSpec((H, DVP), full),
            pl.BlockSpec((CS + DVP, CS), full),
            pl.BlockSpec((CS, CS), full),
            pl.BlockSpec((CS, CS), full),
        ],
        out_specs=pl.BlockSpec((BN_A, CS), row),
        out_shape=jax.ShapeDtypeStruct((N, CS), jnp.float32),
    )(nf, a1, a2, xt, r1, r2, wo, wt1, wt2)


def kernel(node_features, edge_features, edge_index, x_ca, Wq, Wk, Wv,
           Wqp, Wkp, Wvp, Wb, Wo, Wt1, Wt2):
    eib = lax.bitcast_convert_type(
        edge_index.astype(jnp.int32).T, jnp.float32)
    xt = jnp.tile(x_ca, (1, H * PV))
    r1 = jnp.asarray(np.kron(np.eye(H, dtype=np.float32),
                             np.ones((1, CH), np.float32)))
    r2 = jnp.asarray(np.kron(np.eye(H, dtype=np.float32),
                             np.ones((1, PV * 3), np.float32)))
    wqp_pad = jnp.pad(Wqp.reshape(CS, H, PQK * 3),
                      ((0, 0), (0, 0), (0, 4))).reshape(CS, CS)
    wkp_pad = jnp.pad(Wkp.reshape(CS, H, PQK * 3),
                      ((0, 0), (0, 0), (0, 4))).reshape(CS, CS)
    xqh = jnp.concatenate([jnp.tile(x_ca, (1, PQK)),
                           jnp.zeros((N, 4), jnp.float32)], axis=1)
    xqp = jnp.tile(xqh, (1, H))
    gdst, ga, gb = _projections(node_features, xt, xqp, Wq, Wk, Wv,
                                wqp_pad, wkp_pad, Wvp)
    rec = _edge_record(eib, edge_features, Wb)
    rec = jnp.pad(rec, ((0, EPAD - E), (0, 0)))
    a1, a2 = _edge_phase_sc(rec, gdst, ga, gb)
    return _epilogue(node_features, a1, a2, xt, r1, r2, Wo, Wt1, Wt2)


# speedup vs baseline: 3.7523x; 1.8265x over previous
import functools

import jax
import jax.numpy as jnp
import numpy as np
from jax import lax
from jax.experimental import pallas as pl
from jax.experimental.pallas import tpu as pltpu
from jax.experimental.pallas import tpu_sc as plsc

N = 10000
E = 320000
CS = 128
CZ = 128
H = 8
CH = 16
PQK = 4
PV = 8

DQP = H * PQK * 3
DVP = H * PV * 3
DQPP = H * CH
DDST = CS + DQPP
DGA = CS + DQPP + CS
DGB = DVP
DREC = 16
DA1 = 144
DA2 = 192

NSUB = 16
EPT = 20096
EPAD = NSUB * EPT
K1 = 32
K2 = 64
NC1 = EPT // K1
NC2 = EPT // K2
HALF = 5000
RPC = 5120
ROWS_PT = RPC // NSUB

BN_A = 1000
BN_B = 8000


def _proj_body(nf, xt, xqp, wq, wk, wv, wqp, wkp, wvp, gdst, ga, gb):
    x = nf[...]
    xq = xqp[...]
    q = jnp.dot(x, wq[...], preferred_element_type=jnp.float32)
    qp = jnp.dot(x, wqp[...], preferred_element_type=jnp.float32) + xq
    gdst[...] = jnp.concatenate([q, qp], axis=-1)
    k = jnp.dot(x, wk[...], preferred_element_type=jnp.float32)
    kp = jnp.dot(x, wkp[...], preferred_element_type=jnp.float32) + xq
    v = jnp.dot(x, wv[...], preferred_element_type=jnp.float32)
    ga[...] = jnp.concatenate([k, kp, v], axis=-1)
    gb[...] = jnp.dot(x, wvp[...], preferred_element_type=jnp.float32) + xt[...]


def _projections(nf, xt, xqp, wq, wk, wv, wqp, wkp, wvp):
    grid = (N // BN_A,)
    row = lambda i: (i, 0)
    full = lambda i: (0, 0)
    return pl.pallas_call(
        _proj_body,
        grid=grid,
        in_specs=[
            pl.BlockSpec((BN_A, CS), row),
            pl.BlockSpec((BN_A, DVP), row),
            pl.BlockSpec((BN_A, CS), row),
            pl.BlockSpec((CS, CS), full),
            pl.BlockSpec((CS, CS), full),
            pl.BlockSpec((CS, CS), full),
            pl.BlockSpec((CS, CS), full),
            pl.BlockSpec((CS, CS), full),
            pl.BlockSpec((CS, DVP), full),
        ],
        out_specs=[
            pl.BlockSpec((BN_A, DDST), row),
            pl.BlockSpec((BN_A, DGA), row),
            pl.BlockSpec((BN_A, DGB), row),
        ],
        out_shape=[
            jax.ShapeDtypeStruct((N, DDST), jnp.float32),
            jax.ShapeDtypeStruct((N, DGA), jnp.float32),
            jax.ShapeDtypeStruct((N, DGB), jnp.float32),
        ],
    )(nf, xt, xqp, wq, wk, wv, wqp, wkp, wvp)


def _rec_body(eib, ef, wb, out):
    b = jnp.dot(ef[...], wb[...], preferred_element_type=jnp.float32)
    z = jnp.zeros((BN_B, 6), jnp.float32)
    out[...] = jnp.concatenate([eib[...], z, b], axis=-1)


def _edge_record(eib, ef, wb):
    grid = (E // BN_B,)
    return pl.pallas_call(
        _rec_body,
        grid=grid,
        in_specs=[
            pl.BlockSpec((BN_B, 2), lambda i: (i, 0)),
            pl.BlockSpec((BN_B, CZ), lambda i: (i, 0)),
            pl.BlockSpec((CZ, H), lambda i: (0, 0)),
        ],
        out_specs=pl.BlockSpec((BN_B, DREC), lambda i: (i, 0)),
        out_shape=jax.ShapeDtypeStruct((E, DREC), jnp.float32),
    )(eib, ef, wb)


def _sc_mesh():
    return plsc.VectorSubcoreMesh(core_axis_name="c", subcore_axis_name="s")


def _zero_acc(ct, acc_sh, s_idx, kc):
    zero16 = jnp.zeros((16,), jnp.float32)
    width = ct.shape[1]
    for r in range(kc):
        for cc in range(width // 16):
            ct[r, pl.ds(cc * 16, 16)] = zero16
    for off in range(0, ROWS_PT, kc):
        sz = min(kc, ROWS_PT - off)
        pltpu.sync_copy(ct.at[pl.ds(0, sz)],
                        acc_sh.at[pl.ds(s_idx * ROWS_PT + off, sz)])


def _copy_out(acc_sh, out_hbm, c_idx, s_idx):
    pltpu.sync_copy(acc_sh.at[pl.ds(s_idx * ROWS_PT, ROWS_PT)],
                    out_hbm.at[c_idx, pl.ds(s_idx * ROWS_PT, ROWS_PT)])


def _c1_body(rec_hbm, gdst_hbm, ga_hbm, acc_hbm, ex_hbm,
             rec0, rec1, gd0, gd1, ga0, ga1, ct0, ct1, exv0, exv1,
             srcv0, srcv1, dstv0, dstv1, idxm0, idxm1, acc_sh,
             srec0, srec1, sgd0, sgd1, sga0, sga1, ssc0, ssc1, sex0, sex1):
    c_idx = lax.axis_index("c")
    s_idx = lax.axis_index("s")
    lo = c_idx * HALF
    iot = lax.iota(jnp.int32, 16)
    base = s_idx * EPT
    is0 = c_idx == 0

    rec = (rec0, rec1)
    gd = (gd0, gd1)
    ga = (ga0, ga1)
    ct = (ct0, ct1)
    exv = (exv0, exv1)
    srcv = (srcv0, srcv1)
    dstv = (dstv0, dstv1)
    idxm = (idxm0, idxm1)
    srec = (srec0, srec1)
    sgd = (sgd0, sgd1)
    sga = (sga0, sga1)
    ssc = (ssc0, ssc1)
    sex = (sex0, sex1)

    _zero_acc(ct0, acc_sh, s_idx, K1)
    plsc.subcore_barrier()

    def fetch_rec(h, s):
        pltpu.async_copy(rec_hbm.at[pl.ds(base + h * K1, K1)], rec[s], srec[s])

    def extract(h, s):
        e0 = base + h * K1
        for g in range(K1 // 16):
            r = g * 16 + iot
            sf = plsc.load_gather(rec[s], [r, jnp.zeros((16,), jnp.int32)])
            df = plsc.load_gather(rec[s], [r, jnp.full((16,), 1, jnp.int32)])
            sv = plsc.bitcast(sf, jnp.int32)
            dv = plsc.bitcast(df, jnp.int32)
            srcv[s][pl.ds(g * 16, 16)] = sv
            dstv[s][pl.ds(g * 16, 16)] = dv
            loc = dv - lo
            ok = (loc >= 0) & (loc < HALF) & ((iot + (e0 + g * 16)) < E)
            idxm[s][pl.ds(g * 16, 16)] = jnp.where(ok, loc, HALF)
        pltpu.async_copy(gdst_hbm.at[dstv[s]], gd[s], sgd[s])
        pltpu.async_copy(ga_hbm.at[srcv[s]], ga[s], sga[s])

    def compute_pair(h0):
        perms = [jnp.bitwise_xor(iot, k) for k in (8, 4, 2, 1)]

        def bsum(x):
            for p in perms:
                x = x + x.at[p].get(mode="promise_in_bounds")
            return x

        brot = jnp.bitwise_and(iot + 8, 15)

        @plsc.parallel_loop(0, K1, unroll=2)
        def _(e):
            for s in (0, 1):
                brow = rec[s][e, pl.ds(0, 16)]
                lrow = jnp.zeros((16,), jnp.float32)
                for hh in range(H):
                    qh = gd[s][e, pl.ds(hh * CH, 16)]
                    kh = ga[s][e, pl.ds(hh * CH, 16)]
                    dh = (gd[s][e, pl.ds(CS + hh * CH, 16)]
                          - ga[s][e, pl.ds(CS + hh * CH, 16)])
                    m = qh * kh * 0.25 - dh * dh * 0.1
                    lrow = jnp.where(iot == hh, bsum(m), lrow)
                exrow = jnp.exp(lrow + brow.at[brot].get(mode="promise_in_bounds"))
                for hh in range(H):
                    exh = exrow.at[jnp.full((16,), hh, jnp.int32)].get(
                        mode="promise_in_bounds")
                    vh = ga[s][e, pl.ds(CS + DQPP + hh * CH, 16)]
                    ct[s][e, pl.ds(16 + hh * CH, 16)] = exh * vh
                ct[s][e, pl.ds(0, 16)] = exrow
                exv[s][e, pl.ds(0, 16)] = exrow
        for s in (0, 1):
            e0 = base + (h0 + s) * K1
            pltpu.async_copy(ct[s], acc_sh.at[idxm[s]], ssc[s], add=True)

            @pl.when(is0)
            def _(s=s, e0=e0):
                pltpu.async_copy(exv[s], ex_hbm.at[pl.ds(e0, K1)], sex[s])

    fetch_rec(0, 0)
    fetch_rec(1, 1)

    npair = NC1 // 2

    def body(p, carry):
        h0 = 2 * p
        for s in (0, 1):
            h = h0 + s

            @pl.when(p > 0)
            def _(s=s):
                pltpu.make_async_copy(ct[s], acc_sh.at[idxm[s]], ssc[s]).wait()

                @pl.when(is0)
                def _():
                    pltpu.make_async_copy(exv[s], ex_hbm.at[pl.ds(0, K1)],
                                          sex[s]).wait()

            pltpu.make_async_copy(rec_hbm.at[pl.ds(0, K1)], rec[s], srec[s]).wait()
            extract(h, s)
        for s in (0, 1):
            pltpu.make_async_copy(gdst_hbm.at[dstv[s]], gd[s], sgd[s]).wait()
            pltpu.make_async_copy(ga_hbm.at[srcv[s]], ga[s], sga[s]).wait()
        compute_pair(h0)
        for s in (0, 1):

            @pl.when(p < npair - 1)
            def _(h=h0 + s, s=s):
                fetch_rec(h + 2, s)

        return carry

    lax.fori_loop(0, npair, body, 0)
    for s in (0, 1):
        pltpu.make_async_copy(ct[s], acc_sh.at[idxm[s]], ssc[s]).wait()

        @pl.when(is0)
        def _(s=s):
            pltpu.make_async_copy(exv[s], ex_hbm.at[pl.ds(0, K1)], sex[s]).wait()

    plsc.subcore_barrier()
    _copy_out(acc_sh, acc_hbm, c_idx, s_idx)


def _c2_body(rec_hbm, ex_hbm, gb_hbm, acc_hbm,
             rec0, rec1, exv0, exv1, gb0, gb1, ct0, ct1,
             srcv0, srcv1, dstv0, dstv1, idxm0, idxm1, acc_sh,
             srec0, srec1, sev0, sev1, sgb0, sgb1, ssc0, ssc1):
    c_idx = lax.axis_index("c")
    s_idx = lax.axis_index("s")
    lo = c_idx * HALF
    iot = lax.iota(jnp.int32, 16)
    base = s_idx * EPT

    rec = (rec0, rec1)
    exv = (exv0, exv1)
    gb = (gb0, gb1)
    ct = (ct0, ct1)
    srcv = (srcv0, srcv1)
    dstv = (dstv0, dstv1)
    idxm = (idxm0, idxm1)
    srec = (srec0, srec1)
    sev = (sev0, sev1)
    sgb = (sgb0, sgb1)
    ssc = (ssc0, ssc1)

    _zero_acc(ct0, acc_sh, s_idx, K2)
    plsc.subcore_barrier()

    def fetch(h, s):
        pltpu.async_copy(rec_hbm.at[pl.ds(base + h * K2, K2)], rec[s], srec[s])
        pltpu.async_copy(ex_hbm.at[pl.ds(base + h * K2, K2)], exv[s], sev[s])

    def extract(h, s):
        e0 = base + h * K2
        for g in range(K2 // 16):
            r = g * 16 + iot
            df = plsc.load_gather(rec[s], [r, jnp.full((16,), 1, jnp.int32)])
            sf = plsc.load_gather(rec[s], [r, jnp.zeros((16,), jnp.int32)])
            dv = plsc.bitcast(df, jnp.int32)
            srcv[s][pl.ds(g * 16, 16)] = plsc.bitcast(sf, jnp.int32)
            loc = dv - lo
            ok = (loc >= 0) & (loc < HALF) & ((iot + (e0 + g * 16)) < E)
            idxm[s][pl.ds(g * 16, 16)] = jnp.where(ok, loc, HALF)
        pltpu.async_copy(gb_hbm.at[srcv[s]], gb[s], sgb[s])

    def compute_pair():
        @plsc.parallel_loop(0, K2, unroll=2)
        def _(e):
            for s in (0, 1):
                exrow = exv[s][e, pl.ds(0, 16)]
                exs = [jnp.full((16,), exrow[hh], jnp.float32)
                       for hh in range(H)]
                for j in range(DGB // 16):
                    a = (16 * j) // 24
                    b = (16 * j + 15) // 24
                    if a == b:
                        exj = exs[a]
                    else:
                        exj = jnp.where(iot < (24 * (a + 1) - 16 * j),
                                        exs[a], exs[b])
                    ct[s][e, pl.ds(16 * j, 16)] = exj * gb[s][e, pl.ds(16 * j, 16)]
        for s in (0, 1):
            pltpu.async_copy(ct[s], acc_sh.at[idxm[s]], ssc[s], add=True)

    fetch(0, 0)
    fetch(1, 1)
    npair = NC2 // 2

    def body(p, carry):
        h0 = 2 * p
        for s in (0, 1):
            h = h0 + s

            @pl.when(p > 0)
            def _(s=s):
                pltpu.make_async_copy(ct[s], acc_sh.at[idxm[s]], ssc[s]).wait()

            pltpu.make_async_copy(rec_hbm.at[pl.ds(0, K2)], rec[s], srec[s]).wait()
            extract(h, s)
        for s in (0, 1):
            pltpu.make_async_copy(ex_hbm.at[pl.ds(0, K2)], exv[s], sev[s]).wait()
            pltpu.make_async_copy(gb_hbm.at[srcv[s]], gb[s], sgb[s]).wait()
        compute_pair()
        for s in (0, 1):

            @pl.when(p < npair - 1)
            def _(h=h0 + s, s=s):
                fetch(h + 2, s)

        return carry

    lax.fori_loop(0, npair, body, 0)
    for s in (0, 1):
        pltpu.make_async_copy(ct[s], acc_sh.at[idxm[s]], ssc[s]).wait()
    plsc.subcore_barrier()
    _copy_out(acc_sh, acc_hbm, c_idx, s_idx)


def _edge_phase_sc(rec, gdst, ga, gb):
    params = pltpu.CompilerParams(use_tc_tiling_on_sc=False,
                                  needs_layout_passes=False)
    c1 = functools.partial(
        pl.kernel,
        out_type=[jax.ShapeDtypeStruct((2, RPC, DA1), jnp.float32),
                  jax.ShapeDtypeStruct((EPAD, 16), jnp.float32)],
        mesh=_sc_mesh(),
        compiler_params=params,
        scratch_types=(
            [pltpu.VMEM((K1, DREC), jnp.float32)] * 2
            + [pltpu.VMEM((K1, DDST), jnp.float32)] * 2
            + [pltpu.VMEM((K1, DGA), jnp.float32)] * 2
            + [pltpu.VMEM((K1, DA1), jnp.float32)] * 2
            + [pltpu.VMEM((K1, 16), jnp.float32)] * 2
            + [pltpu.VMEM((K1,), jnp.int32)] * 6
            + [pltpu.VMEM_SHARED((RPC, DA1), jnp.float32)]
            + [pltpu.SemaphoreType.DMA] * 10
        ),
    )(_c1_body)
    acc1, exbuf = c1(rec, gdst, ga)

    c2 = functools.partial(
        pl.kernel,
        out_type=jax.ShapeDtypeStruct((2, RPC, DA2), jnp.float32),
        mesh=_sc_mesh(),
        compiler_params=params,
        scratch_types=(
            [pltpu.VMEM((K2, DREC), jnp.float32)] * 2
            + [pltpu.VMEM((K2, 16), jnp.float32)] * 2
            + [pltpu.VMEM((K2, DGB), jnp.float32)] * 2
            + [pltpu.VMEM((K2, DA2), jnp.float32)] * 2
            + [pltpu.VMEM((K2,), jnp.int32)] * 6
            + [pltpu.VMEM_SHARED((RPC, DA2), jnp.float32)]
            + [pltpu.SemaphoreType.DMA] * 8
        ),
    )(_c2_body)
    acc2 = c2(rec, exbuf, gb)

    a1 = jnp.concatenate([acc1[0, :HALF], acc1[1, :HALF]], axis=0)
    a2 = jnp.concatenate([acc2[0, :HALF], acc2[1, :HALF]], axis=0)
    return a1, a2


def _ln(x):
    m = x.mean(-1, keepdims=True)
    v = ((x - m) ** 2).mean(-1, keepdims=True)
    return (x - m) * lax.rsqrt(v + 1e-5)


def _epi_body(nf, a1, a2, xt, r1, r2, wo, wt1, wt2, out):
    den = a1[:, :H]
    dinv = 1.0 / jnp.maximum(den, 1e-30)
    rep1 = jnp.dot(dinv, r1[...], preferred_element_type=jnp.float32)
    rep2 = jnp.dot(dinv, r2[...], preferred_element_type=jnp.float32)
    ov = a1[:, 16:16 + CS] * rep1
    op = a2[...] * rep2 - xt[...]
    u = jnp.concatenate([ov, op], axis=-1)
    o = jnp.dot(u, wo[...], preferred_element_type=jnp.float32)
    s = _ln(nf[...] + o)
    t = jnp.dot(jax.nn.relu(jnp.dot(s, wt1[...], preferred_element_type=jnp.float32)),
                wt2[...], preferred_element_type=jnp.float32)
    out[...] = _ln(s + t)


def _epilogue(nf, a1, a2, xt, r1, r2, wo, wt1, wt2):
    grid = (N // BN_A,)
    row = lambda i: (i, 0)
    full = lambda i: (0, 0)
    return pl.pallas_call(
        _epi_body,
        grid=grid,
        in_specs=[
            pl.BlockSpec((BN_A, CS), row),
            pl.BlockSpec((BN_A, DA1), row),
            pl.BlockSpec((BN_A, DA2), row),
            pl.BlockSpec((BN_A, DVP), row),
            pl.BlockSpec((H, CS), full),
            pl.BlockSpec((H, DVP), full),
            pl.BlockSpec((CS + DVP, CS), full),
            pl.BlockSpec((CS, CS), full),
            pl.BlockSpec((CS, CS), full),
        ],
        out_specs=pl.BlockSpec((BN_A, CS), row),
        out_shape=jax.ShapeDtypeStruct((N, CS), jnp.float32),
    )(nf, a1, a2, xt, r1, r2, wo, wt1, wt2)


def kernel(node_features, edge_features, edge_index, x_ca, Wq, Wk, Wv,
           Wqp, Wkp, Wvp, Wb, Wo, Wt1, Wt2):
    eib = lax.bitcast_convert_type(
        edge_index.astype(jnp.int32).T, jnp.float32)
    xt = jnp.tile(x_ca, (1, H * PV))
    r1 = jnp.asarray(np.kron(np.eye(H, dtype=np.float32),
                             np.ones((1, CH), np.float32)))
    r2 = jnp.asarray(np.kron(np.eye(H, dtype=np.float32),
                             np.ones((1, PV * 3), np.float32)))
    wqp_pad = jnp.pad(Wqp.reshape(CS, H, PQK * 3),
                      ((0, 0), (0, 0), (0, 4))).reshape(CS, CS)
    wkp_pad = jnp.pad(Wkp.reshape(CS, H, PQK * 3),
                      ((0, 0), (0, 0), (0, 4))).reshape(CS, CS)
    xqh = jnp.concatenate([jnp.tile(x_ca, (1, PQK)),
                           jnp.zeros((N, 4), jnp.float32)], axis=1)
    xqp = jnp.tile(xqh, (1, H))
    gdst, ga, gb = _projections(node_features, xt, xqp, Wq, Wk, Wv,
                                wqp_pad, wkp_pad, Wvp)
    rec = _edge_record(eib, edge_features, Wb)
    rec = jnp.pad(rec, ((0, EPAD - E), (0, 0)))
    a1, a2 = _edge_phase_sc(rec, gdst, ga, gb)
    return _epilogue(node_features, a1, a2, xt, r1, r2, Wo, Wt1, Wt2)


# speedup vs baseline: 3.7967x; 1.0118x over previous
import functools

import jax
import jax.numpy as jnp
import numpy as np
from jax import lax
from jax.experimental import pallas as pl
from jax.experimental.pallas import tpu as pltpu
from jax.experimental.pallas import tpu_sc as plsc

N = 10000
E = 320000
CS = 128
CZ = 128
H = 8
CH = 16
PQK = 4
PV = 8

DQP = H * PQK * 3
DVP = H * PV * 3
DQPP = H * CH
DDST = CS + DQPP
DGA = CS + DQPP + CS
DGB = DVP
DREC = 16
DA1 = 144
DA2 = 192

NSUB = 16
EPT1 = 20160
EPT2 = 20096
EPAD = NSUB * EPT1
K1 = 48
K2 = 64
NC1 = EPT1 // K1
NC2 = EPT2 // K2
HALF = 5000
RPC = 5120
ROWS_PT = RPC // NSUB

BN_A = 1000
BN_B = 8000


def _proj_body(nf, xt, xqp, wq, wk, wv, wqp, wkp, wvp, gdst, ga, gb):
    x = nf[...]
    xq = xqp[...]
    q = jnp.dot(x, wq[...], preferred_element_type=jnp.float32)
    qp = jnp.dot(x, wqp[...], preferred_element_type=jnp.float32) + xq
    gdst[...] = jnp.concatenate([q, qp], axis=-1)
    k = jnp.dot(x, wk[...], preferred_element_type=jnp.float32)
    kp = jnp.dot(x, wkp[...], preferred_element_type=jnp.float32) + xq
    v = jnp.dot(x, wv[...], preferred_element_type=jnp.float32)
    ga[...] = jnp.concatenate([k, kp, v], axis=-1)
    gb[...] = jnp.dot(x, wvp[...], preferred_element_type=jnp.float32) + xt[...]


def _projections(nf, xt, xqp, wq, wk, wv, wqp, wkp, wvp):
    grid = (N // BN_A,)
    row = lambda i: (i, 0)
    full = lambda i: (0, 0)
    return pl.pallas_call(
        _proj_body,
        grid=grid,
        in_specs=[
            pl.BlockSpec((BN_A, CS), row),
            pl.BlockSpec((BN_A, DVP), row),
            pl.BlockSpec((BN_A, CS), row),
            pl.BlockSpec((CS, CS), full),
            pl.BlockSpec((CS, CS), full),
            pl.BlockSpec((CS, CS), full),
            pl.BlockSpec((CS, CS), full),
            pl.BlockSpec((CS, CS), full),
            pl.BlockSpec((CS, DVP), full),
        ],
        out_specs=[
            pl.BlockSpec((BN_A, DDST), row),
            pl.BlockSpec((BN_A, DGA), row),
            pl.BlockSpec((BN_A, DGB), row),
        ],
        out_shape=[
            jax.ShapeDtypeStruct((N, DDST), jnp.float32),
            jax.ShapeDtypeStruct((N, DGA), jnp.float32),
            jax.ShapeDtypeStruct((N, DGB), jnp.float32),
        ],
    )(nf, xt, xqp, wq, wk, wv, wqp, wkp, wvp)


def _rec_body(eib, ef, wb, out):
    b = jnp.dot(ef[...], wb[...], preferred_element_type=jnp.float32)
    z = jnp.zeros((BN_B, 6), jnp.float32)
    out[...] = jnp.concatenate([eib[...], z, b], axis=-1)


def _edge_record(eib, ef, wb):
    grid = (E // BN_B,)
    return pl.pallas_call(
        _rec_body,
        grid=grid,
        in_specs=[
            pl.BlockSpec((BN_B, 2), lambda i: (i, 0)),
            pl.BlockSpec((BN_B, CZ), lambda i: (i, 0)),
            pl.BlockSpec((CZ, H), lambda i: (0, 0)),
        ],
        out_specs=pl.BlockSpec((BN_B, DREC), lambda i: (i, 0)),
        out_shape=jax.ShapeDtypeStruct((E, DREC), jnp.float32),
    )(eib, ef, wb)


def _sc_mesh():
    return plsc.VectorSubcoreMesh(core_axis_name="c", subcore_axis_name="s")


def _zero_acc(ct, acc_sh, s_idx, kc):
    zero16 = jnp.zeros((16,), jnp.float32)
    width = ct.shape[1]
    for r in range(kc):
        for cc in range(width // 16):
            ct[r, pl.ds(cc * 16, 16)] = zero16
    for off in range(0, ROWS_PT, kc):
        sz = min(kc, ROWS_PT - off)
        pltpu.sync_copy(ct.at[pl.ds(0, sz)],
                        acc_sh.at[pl.ds(s_idx * ROWS_PT + off, sz)])


def _copy_out(acc_sh, out_hbm, c_idx, s_idx):
    pltpu.sync_copy(acc_sh.at[pl.ds(s_idx * ROWS_PT, ROWS_PT)],
                    out_hbm.at[c_idx, pl.ds(s_idx * ROWS_PT, ROWS_PT)])


def _c1_body(rec_hbm, gdst_hbm, ga_hbm, acc_hbm, ex_hbm,
             rec0, rec1, gd0, gd1, ga0, ga1, ct0, ct1, exv0, exv1,
             srcv0, srcv1, dstv0, dstv1, idxm0, idxm1, acc_sh,
             srec0, srec1, sgd0, sgd1, sga0, sga1, ssc0, ssc1, sex0, sex1):
    c_idx = lax.axis_index("c")
    s_idx = lax.axis_index("s")
    lo = c_idx * HALF
    iot = lax.iota(jnp.int32, 16)
    base = s_idx * EPT1
    is0 = c_idx == 0

    rec = (rec0, rec1)
    gd = (gd0, gd1)
    ga = (ga0, ga1)
    ct = (ct0, ct1)
    exv = (exv0, exv1)
    srcv = (srcv0, srcv1)
    dstv = (dstv0, dstv1)
    idxm = (idxm0, idxm1)
    srec = (srec0, srec1)
    sgd = (sgd0, sgd1)
    sga = (sga0, sga1)
    ssc = (ssc0, ssc1)
    sex = (sex0, sex1)

    _zero_acc(ct0, acc_sh, s_idx, K1)
    plsc.subcore_barrier()

    def fetch_rec(h, s):
        pltpu.async_copy(rec_hbm.at[pl.ds(base + h * K1, K1)], rec[s], srec[s])

    def extract(h, s):
        e0 = base + h * K1
        for g in range(K1 // 16):
            r = g * 16 + iot
            sf = plsc.load_gather(rec[s], [r, jnp.zeros((16,), jnp.int32)])
            df = plsc.load_gather(rec[s], [r, jnp.full((16,), 1, jnp.int32)])
            sv = plsc.bitcast(sf, jnp.int32)
            dv = plsc.bitcast(df, jnp.int32)
            srcv[s][pl.ds(g * 16, 16)] = sv
            dstv[s][pl.ds(g * 16, 16)] = dv
            loc = dv - lo
            ok = (loc >= 0) & (loc < HALF) & ((iot + (e0 + g * 16)) < E)
            idxm[s][pl.ds(g * 16, 16)] = jnp.where(ok, loc, HALF)
        pltpu.async_copy(gdst_hbm.at[dstv[s]], gd[s], sgd[s])
        pltpu.async_copy(ga_hbm.at[srcv[s]], ga[s], sga[s])

    def compute_pair(h0):
        perms = [jnp.bitwise_xor(iot, k) for k in (8, 4, 2, 1)]

        def bsum(x):
            for p in perms:
                x = x + x.at[p].get(mode="promise_in_bounds")
            return x

        brot = jnp.bitwise_and(iot + 8, 15)

        @plsc.parallel_loop(0, K1, unroll=2)
        def _(e):
            for s in (0, 1):
                brow = rec[s][e, pl.ds(0, 16)]
                lrow = jnp.zeros((16,), jnp.float32)
                for hh in range(H):
                    qh = gd[s][e, pl.ds(hh * CH, 16)]
                    kh = ga[s][e, pl.ds(hh * CH, 16)]
                    dh = (gd[s][e, pl.ds(CS + hh * CH, 16)]
                          - ga[s][e, pl.ds(CS + hh * CH, 16)])
                    m = qh * kh - dh * dh
                    lrow = jnp.where(iot == hh, bsum(m), lrow)
                exrow = jnp.exp(lrow + brow.at[brot].get(mode="promise_in_bounds"))
                for hh in range(H):
                    exh = exrow.at[jnp.full((16,), hh, jnp.int32)].get(
                        mode="promise_in_bounds")
                    vh = ga[s][e, pl.ds(CS + DQPP + hh * CH, 16)]
                    ct[s][e, pl.ds(16 + hh * CH, 16)] = exh * vh
                ct[s][e, pl.ds(0, 16)] = exrow
                exv[s][e, pl.ds(0, 16)] = exrow
        for s in (0, 1):
            e0 = base + (h0 + s) * K1
            pltpu.async_copy(ct[s], acc_sh.at[idxm[s]], ssc[s], add=True)

            @pl.when(is0)
            def _(s=s, e0=e0):
                pltpu.async_copy(exv[s], ex_hbm.at[pl.ds(e0, K1)], sex[s])

    fetch_rec(0, 0)
    fetch_rec(1, 1)

    npair = NC1 // 2

    def body(p, carry):
        h0 = 2 * p
        for s in (0, 1):
            h = h0 + s

            @pl.when(p > 0)
            def _(s=s):
                pltpu.make_async_copy(ct[s], acc_sh.at[idxm[s]], ssc[s]).wait()

                @pl.when(is0)
                def _():
                    pltpu.make_async_copy(exv[s], ex_hbm.at[pl.ds(0, K1)],
                                          sex[s]).wait()

            pltpu.make_async_copy(rec_hbm.at[pl.ds(0, K1)], rec[s], srec[s]).wait()
            extract(h, s)
        for s in (0, 1):
            pltpu.make_async_copy(gdst_hbm.at[dstv[s]], gd[s], sgd[s]).wait()
            pltpu.make_async_copy(ga_hbm.at[srcv[s]], ga[s], sga[s]).wait()
        compute_pair(h0)
        for s in (0, 1):

            @pl.when(p < npair - 1)
            def _(h=h0 + s, s=s):
                fetch_rec(h + 2, s)

        return carry

    lax.fori_loop(0, npair, body, 0)
    for s in (0, 1):
        pltpu.make_async_copy(ct[s], acc_sh.at[idxm[s]], ssc[s]).wait()

        @pl.when(is0)
        def _(s=s):
            pltpu.make_async_copy(exv[s], ex_hbm.at[pl.ds(0, K1)], sex[s]).wait()

    plsc.subcore_barrier()
    _copy_out(acc_sh, acc_hbm, c_idx, s_idx)


def _c2_body(rec_hbm, ex_hbm, gb_hbm, acc_hbm,
             rec0, rec1, exv0, exv1, gb0, gb1, ct0, ct1,
             srcv0, srcv1, dstv0, dstv1, idxm0, idxm1, acc_sh,
             srec0, srec1, sev0, sev1, sgb0, sgb1, ssc0, ssc1):
    c_idx = lax.axis_index("c")
    s_idx = lax.axis_index("s")
    lo = c_idx * HALF
    iot = lax.iota(jnp.int32, 16)
    base = s_idx * EPT2

    rec = (rec0, rec1)
    exv = (exv0, exv1)
    gb = (gb0, gb1)
    ct = (ct0, ct1)
    srcv = (srcv0, srcv1)
    dstv = (dstv0, dstv1)
    idxm = (idxm0, idxm1)
    srec = (srec0, srec1)
    sev = (sev0, sev1)
    sgb = (sgb0, sgb1)
    ssc = (ssc0, ssc1)

    _zero_acc(ct0, acc_sh, s_idx, K2)
    plsc.subcore_barrier()

    def fetch(h, s):
        pltpu.async_copy(rec_hbm.at[pl.ds(base + h * K2, K2)], rec[s], srec[s])
        pltpu.async_copy(ex_hbm.at[pl.ds(base + h * K2, K2)], exv[s], sev[s])

    def extract(h, s):
        e0 = base + h * K2
        for g in range(K2 // 16):
            r = g * 16 + iot
            df = plsc.load_gather(rec[s], [r, jnp.full((16,), 1, jnp.int32)])
            sf = plsc.load_gather(rec[s], [r, jnp.zeros((16,), jnp.int32)])
            dv = plsc.bitcast(df, jnp.int32)
            srcv[s][pl.ds(g * 16, 16)] = plsc.bitcast(sf, jnp.int32)
            loc = dv - lo
            ok = (loc >= 0) & (loc < HALF) & ((iot + (e0 + g * 16)) < E)
            idxm[s][pl.ds(g * 16, 16)] = jnp.where(ok, loc, HALF)
        pltpu.async_copy(gb_hbm.at[srcv[s]], gb[s], sgb[s])

    def compute_pair():
        @plsc.parallel_loop(0, K2, unroll=2)
        def _(e):
            for s in (0, 1):
                exrow = exv[s][e, pl.ds(0, 16)]
                exs = [jnp.full((16,), exrow[hh], jnp.float32)
                       for hh in range(H)]
                for j in range(DGB // 16):
                    a = (16 * j) // 24
                    b = (16 * j + 15) // 24
                    if a == b:
                        exj = exs[a]
                    else:
                        exj = jnp.where(iot < (24 * (a + 1) - 16 * j),
                                        exs[a], exs[b])
                    ct[s][e, pl.ds(16 * j, 16)] = exj * gb[s][e, pl.ds(16 * j, 16)]
        for s in (0, 1):
            pltpu.async_copy(ct[s], acc_sh.at[idxm[s]], ssc[s], add=True)

    fetch(0, 0)
    fetch(1, 1)
    npair = NC2 // 2

    def body(p, carry):
        h0 = 2 * p
        for s in (0, 1):
            h = h0 + s

            @pl.when(p > 0)
            def _(s=s):
                pltpu.make_async_copy(ct[s], acc_sh.at[idxm[s]], ssc[s]).wait()

            pltpu.make_async_copy(rec_hbm.at[pl.ds(0, K2)], rec[s], srec[s]).wait()
            extract(h, s)
        for s in (0, 1):
            pltpu.make_async_copy(ex_hbm.at[pl.ds(0, K2)], exv[s], sev[s]).wait()
            pltpu.make_async_copy(gb_hbm.at[srcv[s]], gb[s], sgb[s]).wait()
        compute_pair()
        for s in (0, 1):

            @pl.when(p < npair - 1)
            def _(h=h0 + s, s=s):
                fetch(h + 2, s)

        return carry

    lax.fori_loop(0, npair, body, 0)
    for s in (0, 1):
        pltpu.make_async_copy(ct[s], acc_sh.at[idxm[s]], ssc[s]).wait()
    plsc.subcore_barrier()
    _copy_out(acc_sh, acc_hbm, c_idx, s_idx)


def _edge_phase_sc(rec, gdst, ga, gb):
    params = pltpu.CompilerParams(use_tc_tiling_on_sc=False,
                                  needs_layout_passes=False)
    c1 = functools.partial(
        pl.kernel,
        out_type=[jax.ShapeDtypeStruct((2, RPC, DA1), jnp.float32),
                  jax.ShapeDtypeStruct((EPAD, 16), jnp.float32)],
        mesh=_sc_mesh(),
        compiler_params=params,
        scratch_types=(
            [pltpu.VMEM((K1, DREC), jnp.float32)] * 2
            + [pltpu.VMEM((K1, DDST), jnp.float32)] * 2
            + [pltpu.VMEM((K1, DGA), jnp.float32)] * 2
            + [pltpu.VMEM((K1, DA1), jnp.float32)] * 2
            + [pltpu.VMEM((K1, 16), jnp.float32)] * 2
            + [pltpu.VMEM((K1,), jnp.int32)] * 6
            + [pltpu.VMEM_SHARED((RPC, DA1), jnp.float32)]
            + [pltpu.SemaphoreType.DMA] * 10
        ),
    )(_c1_body)
    acc1, exbuf = c1(rec, gdst, ga)

    c2 = functools.partial(
        pl.kernel,
        out_type=jax.ShapeDtypeStruct((2, RPC, DA2), jnp.float32),
        mesh=_sc_mesh(),
        compiler_params=params,
        scratch_types=(
            [pltpu.VMEM((K2, DREC), jnp.float32)] * 2
            + [pltpu.VMEM((K2, 16), jnp.float32)] * 2
            + [pltpu.VMEM((K2, DGB), jnp.float32)] * 2
            + [pltpu.VMEM((K2, DA2), jnp.float32)] * 2
            + [pltpu.VMEM((K2,), jnp.int32)] * 6
            + [pltpu.VMEM_SHARED((RPC, DA2), jnp.float32)]
            + [pltpu.SemaphoreType.DMA] * 8
        ),
    )(_c2_body)
    acc2 = c2(rec, exbuf, gb)

    a1 = jnp.concatenate([acc1[0, :HALF], acc1[1, :HALF]], axis=0)
    a2 = jnp.concatenate([acc2[0, :HALF], acc2[1, :HALF]], axis=0)
    return a1, a2


def _ln(x):
    m = x.mean(-1, keepdims=True)
    v = ((x - m) ** 2).mean(-1, keepdims=True)
    return (x - m) * lax.rsqrt(v + 1e-5)


def _epi_body(nf, a1, a2, xt, r1, r2, wo, wt1, wt2, out):
    den = a1[:, :H]
    dinv = 1.0 / jnp.maximum(den, 1e-30)
    rep1 = jnp.dot(dinv, r1[...], preferred_element_type=jnp.float32)
    rep2 = jnp.dot(dinv, r2[...], preferred_element_type=jnp.float32)
    ov = a1[:, 16:16 + CS] * rep1
    op = a2[...] * rep2 - xt[...]
    u = jnp.concatenate([ov, op], axis=-1)
    o = jnp.dot(u, wo[...], preferred_element_type=jnp.float32)
    s = _ln(nf[...] + o)
    t = jnp.dot(jax.nn.relu(jnp.dot(s, wt1[...], preferred_element_type=jnp.float32)),
                wt2[...], preferred_element_type=jnp.float32)
    out[...] = _ln(s + t)


def _epilogue(nf, a1, a2, xt, r1, r2, wo, wt1, wt2):
    grid = (N // BN_A,)
    row = lambda i: (i, 0)
    full = lambda i: (0, 0)
    return pl.pallas_call(
        _epi_body,
        grid=grid,
        in_specs=[
            pl.BlockSpec((BN_A, CS), row),
            pl.BlockSpec((BN_A, DA1), row),
            pl.BlockSpec((BN_A, DA2), row),
            pl.BlockSpec((BN_A, DVP), row),
            pl.BlockSpec((H, CS), full),
            pl.BlockSpec((H, DVP), full),
            pl.BlockSpec((CS + DVP, CS), full),
            pl.BlockSpec((CS, CS), full),
            pl.BlockSpec((CS, CS), full),
        ],
        out_specs=pl.BlockSpec((BN_A, CS), row),
        out_shape=jax.ShapeDtypeStruct((N, CS), jnp.float32),
    )(nf, a1, a2, xt, r1, r2, wo, wt1, wt2)


def kernel(node_features, edge_features, edge_index, x_ca, Wq, Wk, Wv,
           Wqp, Wkp, Wvp, Wb, Wo, Wt1, Wt2):
    eib = lax.bitcast_convert_type(
        edge_index.astype(jnp.int32).T, jnp.float32)
    xt = jnp.tile(x_ca, (1, H * PV))
    r1 = jnp.asarray(np.kron(np.eye(H, dtype=np.float32),
                             np.ones((1, CH), np.float32)))
    r2 = jnp.asarray(np.kron(np.eye(H, dtype=np.float32),
                             np.ones((1, PV * 3), np.float32)))
    sp = np.float32(np.sqrt(0.1))
    wqp_pad = jnp.pad(Wqp.reshape(CS, H, PQK * 3),
                      ((0, 0), (0, 0), (0, 4))).reshape(CS, CS) * sp
    wkp_pad = jnp.pad(Wkp.reshape(CS, H, PQK * 3),
                      ((0, 0), (0, 0), (0, 4))).reshape(CS, CS) * sp
    xqh = jnp.concatenate([jnp.tile(x_ca, (1, PQK)),
                           jnp.zeros((N, 4), jnp.float32)], axis=1)
    xqp = jnp.tile(xqh, (1, H)) * sp
    gdst, ga, gb = _projections(node_features, xt, xqp, Wq * 0.25, Wk, Wv,
                                wqp_pad, wkp_pad, Wvp)
    rec = _edge_record(eib, edge_features, Wb)
    rec = jnp.pad(rec, ((0, EPAD - E), (0, 0)))
    a1, a2 = _edge_phase_sc(rec, gdst, ga, gb)
    return _epilogue(node_features, a1, a2, xt, r1, r2, Wo, Wt1, Wt2)


# speedup vs baseline: 3.9637x; 1.0440x over previous
import functools

import jax
import jax.numpy as jnp
import numpy as np
from jax import lax
from jax.experimental import pallas as pl
from jax.experimental.pallas import tpu as pltpu
from jax.experimental.pallas import tpu_sc as plsc

N = 10000
E = 320000
CS = 128
CZ = 128
H = 8
CH = 16
PQK = 4
PV = 8

DQP = H * PQK * 3
DVP = H * PV * 3
DQPP = H * CH
DDST = CS + DQPP
DGA = CS + DQPP + CS
DGB = DVP
DREC = 16
DA1 = 144
DA2 = 192

NSUB = 16
EPT1 = 20160
EPT2 = 20096
EPAD = NSUB * EPT1
K1 = 48
K2 = 64
NC1 = EPT1 // K1
NC2 = EPT2 // K2
HALF = 5000
RPC = 5120
ROWS_PT = RPC // NSUB

BN_A = 1000
BN_B = 8000


def _proj_body(nf, xt, xqp, wq, wk, wv, wqp, wkp, wvp, gdst, ga, gb):
    x = nf[...]
    xq = xqp[...]
    q = jnp.dot(x, wq[...], preferred_element_type=jnp.float32)
    qp = jnp.dot(x, wqp[...], preferred_element_type=jnp.float32) + xq
    gdst[...] = jnp.concatenate([q, qp], axis=-1)
    k = jnp.dot(x, wk[...], preferred_element_type=jnp.float32)
    kp = jnp.dot(x, wkp[...], preferred_element_type=jnp.float32) + xq
    v = jnp.dot(x, wv[...], preferred_element_type=jnp.float32)
    ga[...] = jnp.concatenate([k, kp, v], axis=-1)
    gb[...] = jnp.dot(x, wvp[...], preferred_element_type=jnp.float32) + xt[...]


def _projections(nf, xt, xqp, wq, wk, wv, wqp, wkp, wvp):
    grid = (N // BN_A,)
    row = lambda i: (i, 0)
    full = lambda i: (0, 0)
    return pl.pallas_call(
        _proj_body,
        grid=grid,
        in_specs=[
            pl.BlockSpec((BN_A, CS), row),
            pl.BlockSpec((BN_A, DVP), row),
            pl.BlockSpec((BN_A, CS), row),
            pl.BlockSpec((CS, CS), full),
            pl.BlockSpec((CS, CS), full),
            pl.BlockSpec((CS, CS), full),
            pl.BlockSpec((CS, CS), full),
            pl.BlockSpec((CS, CS), full),
            pl.BlockSpec((CS, DVP), full),
        ],
        out_specs=[
            pl.BlockSpec((BN_A, DDST), row),
            pl.BlockSpec((BN_A, DGA), row),
            pl.BlockSpec((BN_A, DGB), row),
        ],
        out_shape=[
            jax.ShapeDtypeStruct((N, DDST), jnp.float32),
            jax.ShapeDtypeStruct((N, DGA), jnp.float32),
            jax.ShapeDtypeStruct((N, DGB), jnp.float32),
        ],
    )(nf, xt, xqp, wq, wk, wv, wqp, wkp, wvp)


def _rec_body(eib, ef, wb, out):
    b = jnp.dot(ef[...], wb[...], preferred_element_type=jnp.float32)
    z = jnp.zeros((BN_B, 6), jnp.float32)
    out[...] = jnp.concatenate([eib[...], z, b], axis=-1)


def _edge_record(eib, ef, wb):
    grid = (E // BN_B,)
    return pl.pallas_call(
        _rec_body,
        grid=grid,
        in_specs=[
            pl.BlockSpec((BN_B, 2), lambda i: (i, 0)),
            pl.BlockSpec((BN_B, CZ), lambda i: (i, 0)),
            pl.BlockSpec((CZ, H), lambda i: (0, 0)),
        ],
        out_specs=pl.BlockSpec((BN_B, DREC), lambda i: (i, 0)),
        out_shape=jax.ShapeDtypeStruct((E, DREC), jnp.float32),
    )(eib, ef, wb)


def _sc_mesh():
    return plsc.VectorSubcoreMesh(core_axis_name="c", subcore_axis_name="s")


def _zero_acc(ct, acc_sh, s_idx, kc):
    zero16 = jnp.zeros((16,), jnp.float32)
    width = ct.shape[1]
    for r in range(kc):
        for cc in range(width // 16):
            ct[r, pl.ds(cc * 16, 16)] = zero16
    for off in range(0, ROWS_PT, kc):
        sz = min(kc, ROWS_PT - off)
        pltpu.sync_copy(ct.at[pl.ds(0, sz)],
                        acc_sh.at[pl.ds(s_idx * ROWS_PT + off, sz)])


def _copy_out(acc_sh, out_hbm, c_idx, s_idx):
    pltpu.sync_copy(acc_sh.at[pl.ds(s_idx * ROWS_PT, ROWS_PT)],
                    out_hbm.at[c_idx, pl.ds(s_idx * ROWS_PT, ROWS_PT)])


def _c1_body(rec_hbm, gdst_hbm, ga_hbm, acc_hbm, ex_hbm,
             rec0, rec1, gd0, gd1, ga0, ga1, ct0, ct1, exv0, exv1,
             srcv0, srcv1, dstv0, dstv1, idxm0, idxm1, acc_sh,
             srec0, srec1, sgd0, sgd1, sga0, sga1, ssc0, ssc1, sex0, sex1):
    c_idx = lax.axis_index("c")
    s_idx = lax.axis_index("s")
    lo = c_idx * HALF
    iot = lax.iota(jnp.int32, 16)
    base = s_idx * EPT1
    is0 = c_idx == 0

    rec = (rec0, rec1)
    gd = (gd0, gd1)
    ga = (ga0, ga1)
    ct = (ct0, ct1)
    exv = (exv0, exv1)
    srcv = (srcv0, srcv1)
    dstv = (dstv0, dstv1)
    idxm = (idxm0, idxm1)
    srec = (srec0, srec1)
    sgd = (sgd0, sgd1)
    sga = (sga0, sga1)
    ssc = (ssc0, ssc1)
    sex = (sex0, sex1)

    _zero_acc(ct0, acc_sh, s_idx, K1)
    plsc.subcore_barrier()

    def fetch_rec(h, s):
        pltpu.async_copy(rec_hbm.at[pl.ds(base + h * K1, K1)], rec[s], srec[s])

    def extract(h, s):
        e0 = base + h * K1
        for g in range(K1 // 16):
            r = g * 16 + iot
            sf = plsc.load_gather(rec[s], [r, jnp.zeros((16,), jnp.int32)])
            df = plsc.load_gather(rec[s], [r, jnp.full((16,), 1, jnp.int32)])
            sv = plsc.bitcast(sf, jnp.int32)
            dv = plsc.bitcast(df, jnp.int32)
            srcv[s][pl.ds(g * 16, 16)] = sv
            dstv[s][pl.ds(g * 16, 16)] = dv
            loc = dv - lo
            ok = (loc >= 0) & (loc < HALF) & ((iot + (e0 + g * 16)) < E)
            idxm[s][pl.ds(g * 16, 16)] = jnp.where(ok, loc, HALF)
        pltpu.async_copy(gdst_hbm.at[dstv[s]], gd[s], sgd[s])
        pltpu.async_copy(ga_hbm.at[srcv[s]], ga[s], sga[s])

    def compute_pair(h0):
        perms = [jnp.bitwise_xor(iot, k) for k in (8, 4, 2, 1)]

        def bsum(x):
            for p in perms:
                x = x + x.at[p].get(mode="promise_in_bounds")
            return x

        brot = jnp.bitwise_and(iot + 8, 15)

        @plsc.parallel_loop(0, K1, unroll=4)
        def _(e):
            for s in (0, 1):
                brow = rec[s][e, pl.ds(0, 16)]
                lrow = jnp.zeros((16,), jnp.float32)
                for hh in range(H):
                    qh = gd[s][e, pl.ds(hh * CH, 16)]
                    kh = ga[s][e, pl.ds(hh * CH, 16)]
                    dh = (gd[s][e, pl.ds(CS + hh * CH, 16)]
                          - ga[s][e, pl.ds(CS + hh * CH, 16)])
                    m = qh * kh - dh * dh
                    lrow = jnp.where(iot == hh, bsum(m), lrow)
                exrow = jnp.exp(lrow + brow.at[brot].get(mode="promise_in_bounds"))
                for hh in range(H):
                    exh = exrow.at[jnp.full((16,), hh, jnp.int32)].get(
                        mode="promise_in_bounds")
                    vh = ga[s][e, pl.ds(CS + DQPP + hh * CH, 16)]
                    ct[s][e, pl.ds(16 + hh * CH, 16)] = exh * vh
                ct[s][e, pl.ds(0, 16)] = exrow
                exv[s][e, pl.ds(0, 16)] = exrow
        for s in (0, 1):
            e0 = base + (h0 + s) * K1
            pltpu.async_copy(ct[s], acc_sh.at[idxm[s]], ssc[s], add=True)

            @pl.when(is0)
            def _(s=s, e0=e0):
                pltpu.async_copy(exv[s], ex_hbm.at[pl.ds(e0, K1)], sex[s])

    fetch_rec(0, 0)
    fetch_rec(1, 1)

    npair = NC1 // 2

    def body(p, carry):
        h0 = 2 * p
        for s in (0, 1):
            h = h0 + s

            @pl.when(p > 0)
            def _(s=s):
                pltpu.make_async_copy(ct[s], acc_sh.at[idxm[s]], ssc[s]).wait()

                @pl.when(is0)
                def _():
                    pltpu.make_async_copy(exv[s], ex_hbm.at[pl.ds(0, K1)],
                                          sex[s]).wait()

            pltpu.make_async_copy(rec_hbm.at[pl.ds(0, K1)], rec[s], srec[s]).wait()
            extract(h, s)
        for s in (0, 1):
            pltpu.make_async_copy(gdst_hbm.at[dstv[s]], gd[s], sgd[s]).wait()
            pltpu.make_async_copy(ga_hbm.at[srcv[s]], ga[s], sga[s]).wait()
        compute_pair(h0)
        for s in (0, 1):

            @pl.when(p < npair - 1)
            def _(h=h0 + s, s=s):
                fetch_rec(h + 2, s)

        return carry

    lax.fori_loop(0, npair, body, 0)
    for s in (0, 1):
        pltpu.make_async_copy(ct[s], acc_sh.at[idxm[s]], ssc[s]).wait()

        @pl.when(is0)
        def _(s=s):
            pltpu.make_async_copy(exv[s], ex_hbm.at[pl.ds(0, K1)], sex[s]).wait()

    plsc.subcore_barrier()
    _copy_out(acc_sh, acc_hbm, c_idx, s_idx)


def _c2_body(rec_hbm, ex_hbm, gb_hbm, acc_hbm,
             rec0, rec1, exv0, exv1, gb0, gb1, ct0, ct1,
             srcv0, srcv1, dstv0, dstv1, idxm0, idxm1, acc_sh,
             srec0, srec1, sev0, sev1, sgb0, sgb1, ssc0, ssc1):
    c_idx = lax.axis_index("c")
    s_idx = lax.axis_index("s")
    lo = c_idx * HALF
    iot = lax.iota(jnp.int32, 16)
    base = s_idx * EPT2

    rec = (rec0, rec1)
    exv = (exv0, exv1)
    gb = (gb0, gb1)
    ct = (ct0, ct1)
    srcv = (srcv0, srcv1)
    dstv = (dstv0, dstv1)
    idxm = (idxm0, idxm1)
    srec = (srec0, srec1)
    sev = (sev0, sev1)
    sgb = (sgb0, sgb1)
    ssc = (ssc0, ssc1)

    _zero_acc(ct0, acc_sh, s_idx, K2)
    plsc.subcore_barrier()

    def fetch(h, s):
        pltpu.async_copy(rec_hbm.at[pl.ds(base + h * K2, K2)], rec[s], srec[s])
        pltpu.async_copy(ex_hbm.at[pl.ds(base + h * K2, K2)], exv[s], sev[s])

    def extract(h, s):
        e0 = base + h * K2
        for g in range(K2 // 16):
            r = g * 16 + iot
            df = plsc.load_gather(rec[s], [r, jnp.full((16,), 1, jnp.int32)])
            sf = plsc.load_gather(rec[s], [r, jnp.zeros((16,), jnp.int32)])
            dv = plsc.bitcast(df, jnp.int32)
            srcv[s][pl.ds(g * 16, 16)] = plsc.bitcast(sf, jnp.int32)
            loc = dv - lo
            ok = (loc >= 0) & (loc < HALF) & ((iot + (e0 + g * 16)) < E)
            idxm[s][pl.ds(g * 16, 16)] = jnp.where(ok, loc, HALF)
        pltpu.async_copy(gb_hbm.at[srcv[s]], gb[s], sgb[s])

    def compute_pair():
        @plsc.parallel_loop(0, K2, unroll=4)
        def _(e):
            for s in (0, 1):
                exrow = exv[s][e, pl.ds(0, 16)]
                exs = [jnp.full((16,), exrow[hh], jnp.float32)
                       for hh in range(H)]
                for j in range(DGB // 16):
                    a = (16 * j) // 24
                    b = (16 * j + 15) // 24
                    if a == b:
                        exj = exs[a]
                    else:
                        exj = jnp.where(iot < (24 * (a + 1) - 16 * j),
                                        exs[a], exs[b])
                    ct[s][e, pl.ds(16 * j, 16)] = exj * gb[s][e, pl.ds(16 * j, 16)]
        for s in (0, 1):
            pltpu.async_copy(ct[s], acc_sh.at[idxm[s]], ssc[s], add=True)

    fetch(0, 0)
    fetch(1, 1)
    npair = NC2 // 2

    def body(p, carry):
        h0 = 2 * p
        for s in (0, 1):
            h = h0 + s

            @pl.when(p > 0)
            def _(s=s):
                pltpu.make_async_copy(ct[s], acc_sh.at[idxm[s]], ssc[s]).wait()

            pltpu.make_async_copy(rec_hbm.at[pl.ds(0, K2)], rec[s], srec[s]).wait()
            extract(h, s)
        for s in (0, 1):
            pltpu.make_async_copy(ex_hbm.at[pl.ds(0, K2)], exv[s], sev[s]).wait()
            pltpu.make_async_copy(gb_hbm.at[srcv[s]], gb[s], sgb[s]).wait()
        compute_pair()
        for s in (0, 1):

            @pl.when(p < npair - 1)
            def _(h=h0 + s, s=s):
                fetch(h + 2, s)

        return carry

    lax.fori_loop(0, npair, body, 0)
    for s in (0, 1):
        pltpu.make_async_copy(ct[s], acc_sh.at[idxm[s]], ssc[s]).wait()
    plsc.subcore_barrier()
    _copy_out(acc_sh, acc_hbm, c_idx, s_idx)


def _edge_phase_sc(rec, gdst, ga, gb):
    params = pltpu.CompilerParams(use_tc_tiling_on_sc=False,
                                  needs_layout_passes=False)
    c1 = functools.partial(
        pl.kernel,
        out_type=[jax.ShapeDtypeStruct((2, RPC, DA1), jnp.float32),
                  jax.ShapeDtypeStruct((EPAD, 16), jnp.float32)],
        mesh=_sc_mesh(),
        compiler_params=params,
        scratch_types=(
            [pltpu.VMEM((K1, DREC), jnp.float32)] * 2
            + [pltpu.VMEM((K1, DDST), jnp.float32)] * 2
            + [pltpu.VMEM((K1, DGA), jnp.float32)] * 2
            + [pltpu.VMEM((K1, DA1), jnp.float32)] * 2
            + [pltpu.VMEM((K1, 16), jnp.float32)] * 2
            + [pltpu.VMEM((K1,), jnp.int32)] * 6
            + [pltpu.VMEM_SHARED((RPC, DA1), jnp.float32)]
            + [pltpu.SemaphoreType.DMA] * 10
        ),
    )(_c1_body)
    acc1, exbuf = c1(rec, gdst, ga)

    c2 = functools.partial(
        pl.kernel,
        out_type=jax.ShapeDtypeStruct((2, RPC, DA2), jnp.float32),
        mesh=_sc_mesh(),
        compiler_params=params,
        scratch_types=(
            [pltpu.VMEM((K2, DREC), jnp.float32)] * 2
            + [pltpu.VMEM((K2, 16), jnp.float32)] * 2
            + [pltpu.VMEM((K2, DGB), jnp.float32)] * 2
            + [pltpu.VMEM((K2, DA2), jnp.float32)] * 2
            + [pltpu.VMEM((K2,), jnp.int32)] * 6
            + [pltpu.VMEM_SHARED((RPC, DA2), jnp.float32)]
            + [pltpu.SemaphoreType.DMA] * 8
        ),
    )(_c2_body)
    acc2 = c2(rec, exbuf, gb)

    a1 = jnp.concatenate([acc1[0, :HALF], acc1[1, :HALF]], axis=0)
    a2 = jnp.concatenate([acc2[0, :HALF], acc2[1, :HALF]], axis=0)
    return a1, a2


def _ln(x):
    m = x.mean(-1, keepdims=True)
    v = ((x - m) ** 2).mean(-1, keepdims=True)
    return (x - m) * lax.rsqrt(v + 1e-5)


def _epi_body(nf, a1, a2, xt, r1, r2, wo, wt1, wt2, out):
    den = a1[:, :H]
    dinv = 1.0 / jnp.maximum(den, 1e-30)
    rep1 = jnp.dot(dinv, r1[...], preferred_element_type=jnp.float32)
    rep2 = jnp.dot(dinv, r2[...], preferred_element_type=jnp.float32)
    ov = a1[:, 16:16 + CS] * rep1
    op = a2[...] * rep2 - xt[...]
    u = jnp.concatenate([ov, op], axis=-1)
    o = jnp.dot(u, wo[...], preferred_element_type=jnp.float32)
    s = _ln(nf[...] + o)
    t = jnp.dot(jax.nn.relu(jnp.dot(s, wt1[...], preferred_element_type=jnp.float32)),
                wt2[...], preferred_element_type=jnp.float32)
    out[...] = _ln(s + t)


def _epilogue(nf, a1, a2, xt, r1, r2, wo, wt1, wt2):
    grid = (N // BN_A,)
    row = lambda i: (i, 0)
    full = lambda i: (0, 0)
    return pl.pallas_call(
        _epi_body,
        grid=grid,
        in_specs=[
            pl.BlockSpec((BN_A, CS), row),
            pl.BlockSpec((BN_A, DA1), row),
            pl.BlockSpec((BN_A, DA2), row),
            pl.BlockSpec((BN_A, DVP), row),
            pl.BlockSpec((H, CS), full),
            pl.BlockSpec((H, DVP), full),
            pl.BlockSpec((CS + DVP, CS), full),
            pl.BlockSpec((CS, CS), full),
            pl.BlockSpec((CS, CS), full),
        ],
        out_specs=pl.BlockSpec((BN_A, CS), row),
        out_shape=jax.ShapeDtypeStruct((N, CS), jnp.float32),
    )(nf, a1, a2, xt, r1, r2, wo, wt1, wt2)


def kernel(node_features, edge_features, edge_index, x_ca, Wq, Wk, Wv,
           Wqp, Wkp, Wvp, Wb, Wo, Wt1, Wt2):
    eib = lax.bitcast_convert_type(
        edge_index.astype(jnp.int32).T, jnp.float32)
    xt = jnp.tile(x_ca, (1, H * PV))
    r1 = jnp.asarray(np.kron(np.eye(H, dtype=np.float32),
                             np.ones((1, CH), np.float32)))
    r2 = jnp.asarray(np.kron(np.eye(H, dtype=np.float32),
                             np.ones((1, PV * 3), np.float32)))
    sp = np.float32(np.sqrt(0.1))
    wqp_pad = jnp.pad(Wqp.reshape(CS, H, PQK * 3),
                      ((0, 0), (0, 0), (0, 4))).reshape(CS, CS) * sp
    wkp_pad = jnp.pad(Wkp.reshape(CS, H, PQK * 3),
                      ((0, 0), (0, 0), (0, 4))).reshape(CS, CS) * sp
    xqh = jnp.concatenate([jnp.tile(x_ca, (1, PQK)),
                           jnp.zeros((N, 4), jnp.float32)], axis=1)
    xqp = jnp.tile(xqh, (1, H)) * sp
    gdst, ga, gb = _projections(node_features, xt, xqp, Wq * 0.25, Wk, Wv,
                                wqp_pad, wkp_pad, Wvp)
    rec = _edge_record(eib, edge_features, Wb)
    rec = jnp.pad(rec, ((0, EPAD - E), (0, 0)))
    a1, a2 = _edge_phase_sc(rec, gdst, ga, gb)
    return _epilogue(node_features, a1, a2, xt, r1, r2, Wo, Wt1, Wt2)


# speedup vs baseline: 3.9945x; 1.0078x over previous
import functools

import jax
import jax.numpy as jnp
import numpy as np
from jax import lax
from jax.experimental import pallas as pl
from jax.experimental.pallas import tpu as pltpu
from jax.experimental.pallas import tpu_sc as plsc

N = 10000
E = 320000
CS = 128
CZ = 128
H = 8
CH = 16
PQK = 4
PV = 8

DQP = H * PQK * 3
DVP = H * PV * 3
DQPP = H * CH
DDST = CS + DQPP
DGA = CS + DQPP + CS
DGB = DVP
DREC = 16
DA1 = 144
DA2 = 192

NSUB = 16
EPT1 = 20160
EPT2 = 20096
EPAD = NSUB * EPT1
K1 = 48
K2 = 64
NC1 = EPT1 // K1
NC2 = EPT2 // K2
HALF = 5000
RPC = 5120
ROWS_PT = RPC // NSUB

BN_A = 1000
BN_B = 8000


def _proj_body(nf, xt, xqp, wq, wk, wv, wqp, wkp, wvp, gdst, ga, gb):
    x = nf[...]
    xq = xqp[...]
    q = jnp.dot(x, wq[...], preferred_element_type=jnp.float32)
    qp = jnp.dot(x, wqp[...], preferred_element_type=jnp.float32) + xq
    gdst[...] = jnp.concatenate([q, qp], axis=-1)
    k = jnp.dot(x, wk[...], preferred_element_type=jnp.float32)
    kp = jnp.dot(x, wkp[...], preferred_element_type=jnp.float32) + xq
    v = jnp.dot(x, wv[...], preferred_element_type=jnp.float32)
    ga[...] = jnp.concatenate([k, kp, v], axis=-1)
    gb[...] = jnp.dot(x, wvp[...], preferred_element_type=jnp.float32) + xt[...]


def _projections(nf, xt, xqp, wq, wk, wv, wqp, wkp, wvp):
    grid = (N // BN_A,)
    row = lambda i: (i, 0)
    full = lambda i: (0, 0)
    return pl.pallas_call(
        _proj_body,
        grid=grid,
        in_specs=[
            pl.BlockSpec((BN_A, CS), row),
            pl.BlockSpec((BN_A, DVP), row),
            pl.BlockSpec((BN_A, CS), row),
            pl.BlockSpec((CS, CS), full),
            pl.BlockSpec((CS, CS), full),
            pl.BlockSpec((CS, CS), full),
            pl.BlockSpec((CS, CS), full),
            pl.BlockSpec((CS, CS), full),
            pl.BlockSpec((CS, DVP), full),
        ],
        out_specs=[
            pl.BlockSpec((BN_A, DDST), row),
            pl.BlockSpec((BN_A, DGA), row),
            pl.BlockSpec((BN_A, DGB), row),
        ],
        out_shape=[
            jax.ShapeDtypeStruct((N, DDST), jnp.float32),
            jax.ShapeDtypeStruct((N, DGA), jnp.float32),
            jax.ShapeDtypeStruct((N, DGB), jnp.float32),
        ],
    )(nf, xt, xqp, wq, wk, wv, wqp, wkp, wvp)


def _rec_body(eib, ef, wb, out):
    b = jnp.dot(ef[...], wb[...], preferred_element_type=jnp.float32)
    z = jnp.zeros((BN_B, 6), jnp.float32)
    out[...] = jnp.concatenate([eib[...], z, b], axis=-1)


def _edge_record(eib, ef, wb):
    grid = (E // BN_B,)
    return pl.pallas_call(
        _rec_body,
        grid=grid,
        in_specs=[
            pl.BlockSpec((BN_B, 2), lambda i: (i, 0)),
            pl.BlockSpec((BN_B, CZ), lambda i: (i, 0)),
            pl.BlockSpec((CZ, H), lambda i: (0, 0)),
        ],
        out_specs=pl.BlockSpec((BN_B, DREC), lambda i: (i, 0)),
        out_shape=jax.ShapeDtypeStruct((E, DREC), jnp.float32),
    )(eib, ef, wb)


def _sc_mesh():
    return plsc.VectorSubcoreMesh(core_axis_name="c", subcore_axis_name="s")


def _zero_acc(ct, acc_sh, s_idx, kc):
    zero16 = jnp.zeros((16,), jnp.float32)
    width = ct.shape[1]
    for r in range(kc):
        for cc in range(width // 16):
            ct[r, pl.ds(cc * 16, 16)] = zero16
    for off in range(0, ROWS_PT, kc):
        sz = min(kc, ROWS_PT - off)
        pltpu.sync_copy(ct.at[pl.ds(0, sz)],
                        acc_sh.at[pl.ds(s_idx * ROWS_PT + off, sz)])


def _copy_out(acc_sh, out_hbm, c_idx, s_idx):
    pltpu.sync_copy(acc_sh.at[pl.ds(s_idx * ROWS_PT, ROWS_PT)],
                    out_hbm.at[c_idx, pl.ds(s_idx * ROWS_PT, ROWS_PT)])


def _c1_body(rec_hbm, gdst_hbm, ga_hbm, acc_hbm, ex_hbm,
             rec0, rec1, gd0, gd1, ga0, ga1, ct0, ct1, exv0, exv1,
             srcv0, srcv1, dstv0, dstv1, idxm0, idxm1, acc_sh,
             srec0, srec1, sgd0, sgd1, sga0, sga1, ssc0, ssc1, sex0, sex1):
    c_idx = lax.axis_index("c")
    s_idx = lax.axis_index("s")
    lo = c_idx * HALF
    iot = lax.iota(jnp.int32, 16)
    base = s_idx * EPT1
    is0 = c_idx == 0

    rec = (rec0, rec1)
    gd = (gd0, gd1)
    ga = (ga0, ga1)
    ct = (ct0, ct1)
    exv = (exv0, exv1)
    srcv = (srcv0, srcv1)
    dstv = (dstv0, dstv1)
    idxm = (idxm0, idxm1)
    srec = (srec0, srec1)
    sgd = (sgd0, sgd1)
    sga = (sga0, sga1)
    ssc = (ssc0, ssc1)
    sex = (sex0, sex1)

    _zero_acc(ct0, acc_sh, s_idx, K1)
    plsc.subcore_barrier()

    def fetch_rec(h, s):
        pltpu.async_copy(rec_hbm.at[pl.ds(base + h * K1, K1)], rec[s], srec[s])

    def extract(h, s):
        e0 = base + h * K1
        for g in range(K1 // 16):
            r = g * 16 + iot
            sf = plsc.load_gather(rec[s], [r, jnp.zeros((16,), jnp.int32)])
            df = plsc.load_gather(rec[s], [r, jnp.full((16,), 1, jnp.int32)])
            sv = plsc.bitcast(sf, jnp.int32)
            dv = plsc.bitcast(df, jnp.int32)
            srcv[s][pl.ds(g * 16, 16)] = sv
            dstv[s][pl.ds(g * 16, 16)] = dv
            loc = dv - lo
            ok = (loc >= 0) & (loc < HALF) & ((iot + (e0 + g * 16)) < E)
            idxm[s][pl.ds(g * 16, 16)] = jnp.where(ok, loc, HALF)
        pltpu.async_copy(gdst_hbm.at[dstv[s]], gd[s], sgd[s])
        pltpu.async_copy(ga_hbm.at[srcv[s]], ga[s], sga[s])

    def compute_pair(h0):
        perms = [jnp.bitwise_xor(iot, k) for k in (8, 4, 2, 1)]

        def bsum(x):
            for p in perms:
                x = x + x.at[p].get(mode="promise_in_bounds")
            return x

        brot = jnp.bitwise_and(iot + 8, 15)

        @plsc.parallel_loop(0, K1, unroll=8)
        def _(e):
            for s in (0, 1):
                brow = rec[s][e, pl.ds(0, 16)]
                lrow = jnp.zeros((16,), jnp.float32)
                for hh in range(H):
                    qh = gd[s][e, pl.ds(hh * CH, 16)]
                    kh = ga[s][e, pl.ds(hh * CH, 16)]
                    dh = (gd[s][e, pl.ds(CS + hh * CH, 16)]
                          - ga[s][e, pl.ds(CS + hh * CH, 16)])
                    m = qh * kh - dh * dh
                    lrow = jnp.where(iot == hh, bsum(m), lrow)
                exrow = jnp.exp(lrow + brow.at[brot].get(mode="promise_in_bounds"))
                for hh in range(H):
                    exh = exrow.at[jnp.full((16,), hh, jnp.int32)].get(
                        mode="promise_in_bounds")
                    vh = ga[s][e, pl.ds(CS + DQPP + hh * CH, 16)]
                    ct[s][e, pl.ds(16 + hh * CH, 16)] = exh * vh
                ct[s][e, pl.ds(0, 16)] = exrow
                exv[s][e, pl.ds(0, 16)] = exrow
        for s in (0, 1):
            e0 = base + (h0 + s) * K1
            pltpu.async_copy(ct[s], acc_sh.at[idxm[s]], ssc[s], add=True)

            @pl.when(is0)
            def _(s=s, e0=e0):
                pltpu.async_copy(exv[s], ex_hbm.at[pl.ds(e0, K1)], sex[s])

    fetch_rec(0, 0)
    fetch_rec(1, 1)

    npair = NC1 // 2

    def body(p, carry):
        h0 = 2 * p
        for s in (0, 1):
            h = h0 + s

            @pl.when(p > 0)
            def _(s=s):
                pltpu.make_async_copy(ct[s], acc_sh.at[idxm[s]], ssc[s]).wait()

                @pl.when(is0)
                def _():
                    pltpu.make_async_copy(exv[s], ex_hbm.at[pl.ds(0, K1)],
                                          sex[s]).wait()

            pltpu.make_async_copy(rec_hbm.at[pl.ds(0, K1)], rec[s], srec[s]).wait()
            extract(h, s)
        for s in (0, 1):
            pltpu.make_async_copy(gdst_hbm.at[dstv[s]], gd[s], sgd[s]).wait()
            pltpu.make_async_copy(ga_hbm.at[srcv[s]], ga[s], sga[s]).wait()
        compute_pair(h0)
        for s in (0, 1):

            @pl.when(p < npair - 1)
            def _(h=h0 + s, s=s):
                fetch_rec(h + 2, s)

        return carry

    lax.fori_loop(0, npair, body, 0)
    for s in (0, 1):
        pltpu.make_async_copy(ct[s], acc_sh.at[idxm[s]], ssc[s]).wait()

        @pl.when(is0)
        def _(s=s):
            pltpu.make_async_copy(exv[s], ex_hbm.at[pl.ds(0, K1)], sex[s]).wait()

    plsc.subcore_barrier()
    _copy_out(acc_sh, acc_hbm, c_idx, s_idx)


def _c2_body(rec_hbm, ex_hbm, gb_hbm, acc_hbm,
             rec0, rec1, exv0, exv1, gb0, gb1, ct0, ct1,
             srcv0, srcv1, dstv0, dstv1, idxm0, idxm1, acc_sh,
             srec0, srec1, sev0, sev1, sgb0, sgb1, ssc0, ssc1):
    c_idx = lax.axis_index("c")
    s_idx = lax.axis_index("s")
    lo = c_idx * HALF
    iot = lax.iota(jnp.int32, 16)
    base = s_idx * EPT2

    rec = (rec0, rec1)
    exv = (exv0, exv1)
    gb = (gb0, gb1)
    ct = (ct0, ct1)
    srcv = (srcv0, srcv1)
    dstv = (dstv0, dstv1)
    idxm = (idxm0, idxm1)
    srec = (srec0, srec1)
    sev = (sev0, sev1)
    sgb = (sgb0, sgb1)
    ssc = (ssc0, ssc1)

    _zero_acc(ct0, acc_sh, s_idx, K2)
    plsc.subcore_barrier()

    def fetch(h, s):
        pltpu.async_copy(rec_hbm.at[pl.ds(base + h * K2, K2)], rec[s], srec[s])
        pltpu.async_copy(ex_hbm.at[pl.ds(base + h * K2, K2)], exv[s], sev[s])

    def extract(h, s):
        e0 = base + h * K2
        for g in range(K2 // 16):
            r = g * 16 + iot
            df = plsc.load_gather(rec[s], [r, jnp.full((16,), 1, jnp.int32)])
            sf = plsc.load_gather(rec[s], [r, jnp.zeros((16,), jnp.int32)])
            dv = plsc.bitcast(df, jnp.int32)
            srcv[s][pl.ds(g * 16, 16)] = plsc.bitcast(sf, jnp.int32)
            loc = dv - lo
            ok = (loc >= 0) & (loc < HALF) & ((iot + (e0 + g * 16)) < E)
            idxm[s][pl.ds(g * 16, 16)] = jnp.where(ok, loc, HALF)
        pltpu.async_copy(gb_hbm.at[srcv[s]], gb[s], sgb[s])

    def compute_pair():
        @plsc.parallel_loop(0, K2, unroll=8)
        def _(e):
            for s in (0, 1):
                exrow = exv[s][e, pl.ds(0, 16)]
                exs = [jnp.full((16,), exrow[hh], jnp.float32)
                       for hh in range(H)]
                for j in range(DGB // 16):
                    a = (16 * j) // 24
                    b = (16 * j + 15) // 24
                    if a == b:
                        exj = exs[a]
                    else:
                        exj = jnp.where(iot < (24 * (a + 1) - 16 * j),
                                        exs[a], exs[b])
                    ct[s][e, pl.ds(16 * j, 16)] = exj * gb[s][e, pl.ds(16 * j, 16)]
        for s in (0, 1):
            pltpu.async_copy(ct[s], acc_sh.at[idxm[s]], ssc[s], add=True)

    fetch(0, 0)
    fetch(1, 1)
    npair = NC2 // 2

    def body(p, carry):
        h0 = 2 * p
        for s in (0, 1):
            h = h0 + s

            @pl.when(p > 0)
            def _(s=s):
                pltpu.make_async_copy(ct[s], acc_sh.at[idxm[s]], ssc[s]).wait()

            pltpu.make_async_copy(rec_hbm.at[pl.ds(0, K2)], rec[s], srec[s]).wait()
            extract(h, s)
        for s in (0, 1):
            pltpu.make_async_copy(ex_hbm.at[pl.ds(0, K2)], exv[s], sev[s]).wait()
            pltpu.make_async_copy(gb_hbm.at[srcv[s]], gb[s], sgb[s]).wait()
        compute_pair()
        for s in (0, 1):

            @pl.when(p < npair - 1)
            def _(h=h0 + s, s=s):
                fetch(h + 2, s)

        return carry

    lax.fori_loop(0, npair, body, 0)
    for s in (0, 1):
        pltpu.make_async_copy(ct[s], acc_sh.at[idxm[s]], ssc[s]).wait()
    plsc.subcore_barrier()
    _copy_out(acc_sh, acc_hbm, c_idx, s_idx)


def _edge_phase_sc(rec, gdst, ga, gb):
    params = pltpu.CompilerParams(use_tc_tiling_on_sc=False,
                                  needs_layout_passes=False)
    c1 = functools.partial(
        pl.kernel,
        out_type=[jax.ShapeDtypeStruct((2, RPC, DA1), jnp.float32),
                  jax.ShapeDtypeStruct((EPAD, 16), jnp.float32)],
        mesh=_sc_mesh(),
        compiler_params=params,
        scratch_types=(
            [pltpu.VMEM((K1, DREC), jnp.float32)] * 2
            + [pltpu.VMEM((K1, DDST), jnp.float32)] * 2
            + [pltpu.VMEM((K1, DGA), jnp.float32)] * 2
            + [pltpu.VMEM((K1, DA1), jnp.float32)] * 2
            + [pltpu.VMEM((K1, 16), jnp.float32)] * 2
            + [pltpu.VMEM((K1,), jnp.int32)] * 6
            + [pltpu.VMEM_SHARED((RPC, DA1), jnp.float32)]
            + [pltpu.SemaphoreType.DMA] * 10
        ),
    )(_c1_body)
    acc1, exbuf = c1(rec, gdst, ga)

    c2 = functools.partial(
        pl.kernel,
        out_type=jax.ShapeDtypeStruct((2, RPC, DA2), jnp.float32),
        mesh=_sc_mesh(),
        compiler_params=params,
        scratch_types=(
            [pltpu.VMEM((K2, DREC), jnp.float32)] * 2
            + [pltpu.VMEM((K2, 16), jnp.float32)] * 2
            + [pltpu.VMEM((K2, DGB), jnp.float32)] * 2
            + [pltpu.VMEM((K2, DA2), jnp.float32)] * 2
            + [pltpu.VMEM((K2,), jnp.int32)] * 6
            + [pltpu.VMEM_SHARED((RPC, DA2), jnp.float32)]
            + [pltpu.SemaphoreType.DMA] * 8
        ),
    )(_c2_body)
    acc2 = c2(rec, exbuf, gb)

    a1 = jnp.concatenate([acc1[0, :HALF], acc1[1, :HALF]], axis=0)
    a2 = jnp.concatenate([acc2[0, :HALF], acc2[1, :HALF]], axis=0)
    return a1, a2


def _ln(x):
    m = x.mean(-1, keepdims=True)
    v = ((x - m) ** 2).mean(-1, keepdims=True)
    return (x - m) * lax.rsqrt(v + 1e-5)


def _epi_body(nf, a1, a2, xt, r1, r2, wo, wt1, wt2, out):
    den = a1[:, :H]
    dinv = 1.0 / jnp.maximum(den, 1e-30)
    rep1 = jnp.dot(dinv, r1[...], preferred_element_type=jnp.float32)
    rep2 = jnp.dot(dinv, r2[...], preferred_element_type=jnp.float32)
    ov = a1[:, 16:16 + CS] * rep1
    op = a2[...] * rep2 - xt[...]
    u = jnp.concatenate([ov, op], axis=-1)
    o = jnp.dot(u, wo[...], preferred_element_type=jnp.float32)
    s = _ln(nf[...] + o)
    t = jnp.dot(jax.nn.relu(jnp.dot(s, wt1[...], preferred_element_type=jnp.float32)),
                wt2[...], preferred_element_type=jnp.float32)
    out[...] = _ln(s + t)


def _epilogue(nf, a1, a2, xt, r1, r2, wo, wt1, wt2):
    grid = (N // BN_A,)
    row = lambda i: (i, 0)
    full = lambda i: (0, 0)
    return pl.pallas_call(
        _epi_body,
        grid=grid,
        in_specs=[
            pl.BlockSpec((BN_A, CS), row),
            pl.BlockSpec((BN_A, DA1), row),
            pl.BlockSpec((BN_A, DA2), row),
            pl.BlockSpec((BN_A, DVP), row),
            pl.BlockSpec((H, CS), full),
            pl.BlockSpec((H, DVP), full),
            pl.BlockSpec((CS + DVP, CS), full),
            pl.BlockSpec((CS, CS), full),
            pl.BlockSpec((CS, CS), full),
        ],
        out_specs=pl.BlockSpec((BN_A, CS), row),
        out_shape=jax.ShapeDtypeStruct((N, CS), jnp.float32),
    )(nf, a1, a2, xt, r1, r2, wo, wt1, wt2)


def kernel(node_features, edge_features, edge_index, x_ca, Wq, Wk, Wv,
           Wqp, Wkp, Wvp, Wb, Wo, Wt1, Wt2):
    eib = lax.bitcast_convert_type(
        edge_index.astype(jnp.int32).T, jnp.float32)
    xt = jnp.tile(x_ca, (1, H * PV))
    r1 = jnp.asarray(np.kron(np.eye(H, dtype=np.float32),
                             np.ones((1, CH), np.float32)))
    r2 = jnp.asarray(np.kron(np.eye(H, dtype=np.float32),
                             np.ones((1, PV * 3), np.float32)))
    sp = np.float32(np.sqrt(0.1))
    wqp_pad = jnp.pad(Wqp.reshape(CS, H, PQK * 3),
                      ((0, 0), (0, 0), (0, 4))).reshape(CS, CS) * sp
    wkp_pad = jnp.pad(Wkp.reshape(CS, H, PQK * 3),
                      ((0, 0), (0, 0), (0, 4))).reshape(CS, CS) * sp
    xqh = jnp.concatenate([jnp.tile(x_ca, (1, PQK)),
                           jnp.zeros((N, 4), jnp.float32)], axis=1)
    xqp = jnp.tile(xqh, (1, H)) * sp
    gdst, ga, gb = _projections(node_features, xt, xqp, Wq * 0.25, Wk, Wv,
                                wqp_pad, wkp_pad, Wvp)
    rec = _edge_record(eib, edge_features, Wb)
    rec = jnp.pad(rec, ((0, EPAD - E), (0, 0)))
    a1, a2 = _edge_phase_sc(rec, gdst, ga, gb)
    return _epilogue(node_features, a1, a2, xt, r1, r2, Wo, Wt1, Wt2)
